# Initial kernel scaffold; baseline (speedup 1.0000x reference)
import jax
import jax.numpy as jnp
from jax.experimental import pallas as pl


def kernel(x, edge_index, edge_weight, W_z, b_z, W_r, b_r, W_h, b_h, W_lin, b_lin):
    raise NotImplementedError("write your pallas kernel here")


# speedup vs baseline: 2.2806x; 2.2806x over previous
import functools

import jax
import jax.numpy as jnp
from jax.experimental import pallas as pl

N = 10000
E = 320000
D = 128
HID = 64
K = 16
PRE_LEN = 4

_BLK = 1000


def _head_body(t_ref, w_ref, bz_ref, bh_ref, wl_ref, bl_ref, o_ref):
    g = jnp.dot(t_ref[...], w_ref[...], preferred_element_type=jnp.float32)
    z = jax.nn.sigmoid(g[:, :HID] + bz_ref[...])
    ht = jnp.tanh(g[:, HID:] + bh_ref[...])
    h = jax.nn.relu((1.0 - z) * ht)
    o_ref[...] = jnp.dot(h, wl_ref[...], preferred_element_type=jnp.float32) + bl_ref[...]


def _head(t_all, w_all, b_z, b_h, w_lin_pad, b_lin_pad):
    grid = N // _BLK
    return pl.pallas_call(
        _head_body,
        grid=(grid,),
        in_specs=[
            pl.BlockSpec((_BLK, 2 * K * D), lambda i: (i, 0)),
            pl.BlockSpec((2 * K * D, D), lambda i: (0, 0)),
            pl.BlockSpec((HID,), lambda i: (0,)),
            pl.BlockSpec((HID,), lambda i: (0,)),
            pl.BlockSpec((HID, 128), lambda i: (0, 0)),
            pl.BlockSpec((128,), lambda i: (0,)),
        ],
        out_specs=pl.BlockSpec((_BLK, 128), lambda i: (i, 0)),
        out_shape=jax.ShapeDtypeStruct((N, 128), jnp.float32),
    )(t_all, w_all, b_z, b_h, w_lin_pad, b_lin_pad)


def kernel(x, edge_index, edge_weight, W_z, b_z, W_r, b_r, W_h, b_h, W_lin, b_lin):
    row, col = edge_index[0], edge_index[1]
    deg_out = jax.ops.segment_sum(edge_weight, row, num_segments=N)
    deg_in = jax.ops.segment_sum(edge_weight, col, num_segments=N)
    norm_out = edge_weight / deg_out[row]
    norm_in = edge_weight / deg_in[col]

    def prop(v, src, dst, nrm):
        return jax.ops.segment_sum(nrm[:, None] * v[src], dst, num_segments=N)

    terms = []
    t0o = t0i = x
    t1o = prop(x, col, row, norm_out)
    t1i = prop(x, row, col, norm_in)
    terms.append((x, x))
    terms.append((t1o, t1i))
    for _ in range(2, K):
        t2o = 2.0 * prop(t1o, col, row, norm_out) - t0o
        t2i = 2.0 * prop(t1i, row, col, norm_in) - t0i
        terms.append((t2o, t2i))
        t0o, t1o = t1o, t2o
        t0i, t1i = t1i, t2i

    t_all = jnp.concatenate([t for pair in terms for t in pair], axis=1)

    wz = W_z[:, :, :D, :]
    wh = W_h[:, :, :D, :]
    w_cat = jnp.concatenate([wz, wh], axis=-1)
    w_all = w_cat.transpose(1, 0, 2, 3).reshape(2 * K * D, 2 * HID)

    w_lin_pad = jnp.zeros((HID, 128), jnp.float32).at[:, :PRE_LEN].set(W_lin)
    b_lin_pad = jnp.zeros((128,), jnp.float32).at[:PRE_LEN].set(b_lin)

    out = _head(t_all, w_all, b_z, b_h, w_lin_pad, b_lin_pad)
    return out[:, :PRE_LEN]


# speedup vs baseline: 6.1405x; 2.6926x over previous
import functools

import jax
import jax.numpy as jnp
from jax import lax
from jax.experimental.compute_on import compute_on
from jax.experimental import pallas as pl
from jax.experimental.pallas import tpu as pltpu
from jax.experimental.pallas import tpu_sc as plsc

N = 10000
E = 320000
D = 128
HID = 64
K = 16
PRE_LEN = 4

NC = 2
NS = 16
L = 16

EPT = E // NS
OCH = 2000
NOCH = EPT // OCH
SUB = 128
NSUB = 15
TAILSZ = OCH - NSUB * SUB
ROWS = 640
RSTEP = 624
ECH = 64

_MESH = plsc.VectorSubcoreMesh(core_axis_name="c", subcore_axis_name="s",
                               num_cores=NC, num_subcores=NS)


def _zero_rows(buf, nrows):
    z = jnp.zeros((L,), jnp.float32)

    def body(i, _):
        r = i // (D // L)
        c = (i % (D // L)) * L
        buf[r, pl.ds(c, L)] = z
        return 0

    lax.fori_loop(0, nrows * (D // L), body, 0)


def _norm_body(ei, w, norm_out, deg_out, zb, wb, nb, d128, d80, dgb, sem,
               deg_sp):
    cidx = lax.axis_index("c")
    sid = lax.axis_index("s")
    off_n = sid * RSTEP

    def zb_body(g, _):
        zb[pl.ds(g * L, L)] = jnp.zeros((L,), jnp.float32)
        return 0

    lax.fori_loop(0, ROWS // L, zb_body, 0)
    pltpu.sync_copy(zb.at[pl.ds(0, ROWS)], deg_sp.at[pl.ds(off_n, ROWS)])
    plsc.subcore_barrier()

    def ch_body(oc, _):
        base = sid * EPT + oc * OCH
        pltpu.sync_copy(w.at[pl.ds(base, OCH)], wb)
        for j in range(NSUB + 1):
            sz = SUB if j < NSUB else TAILSZ
            dbuf = d128 if j < NSUB else d80
            pltpu.sync_copy(ei.at[pl.ds(cidx * E + base + j * SUB, sz)],
                            dbuf)
            pltpu.sync_copy(wb.at[pl.ds(j * SUB, sz)], deg_sp.at[dbuf],
                            add=True)
        return 0

    lax.fori_loop(0, NOCH, ch_body, 0)
    plsc.subcore_barrier()

    pltpu.sync_copy(deg_sp.at[pl.ds(off_n, ROWS)], zb.at[pl.ds(0, ROWS)])
    pltpu.sync_copy(zb.at[pl.ds(0, ROWS)],
                    deg_out.at[pl.ds(cidx * N + off_n, ROWS)])
    plsc.subcore_barrier()

    def nch_body(oc, _):
        base = sid * EPT + oc * OCH
        pltpu.sync_copy(w.at[pl.ds(base, OCH)], wb)
        for j in range(NSUB + 1):
            sz = SUB if j < NSUB else TAILSZ
            dbuf = d128 if j < NSUB else d80
            pltpu.sync_copy(ei.at[pl.ds(cidx * E + base + j * SUB, sz)],
                            dbuf)

            def adj_body(g, _, dbuf=dbuf):
                dbuf[pl.ds(g * L, L)] = dbuf[pl.ds(g * L, L)] + cidx * N
                return 0

            lax.fori_loop(0, sz // L, adj_body, 0)
            pltpu.async_copy(deg_out.at[dbuf], dgb.at[pl.ds(0, sz)],
                             sem).wait()

            def g_body(g, _, j=j):
                wv = wb[pl.ds(j * SUB + g * L, L)]
                dv = dgb[pl.ds(g * L, L)]
                nb[pl.ds(j * SUB + g * L, L)] = wv / dv
                return 0

            lax.fori_loop(0, sz // L, g_body, 0)
        pltpu.sync_copy(nb, norm_out.at[pl.ds(cidx * E + base, OCH)])
        return 0

    lax.fori_loop(0, NOCH, nch_body, 0)


def _norm_call(ei, w):
    with compute_on("tpu_sparsecore"):
        return _norm_call_inner(ei, w)


def _norm_call_inner(ei, w):
    norm, _deg = pl.kernel(
        _norm_body,
        out_type=(jax.ShapeDtypeStruct((NC * E,), jnp.float32),
                  jax.ShapeDtypeStruct((NC * N,), jnp.float32)),
        mesh=_MESH,
        scratch_types=[
            pltpu.VMEM((OCH,), jnp.float32),
            pltpu.VMEM((OCH,), jnp.float32),
            pltpu.VMEM((OCH,), jnp.float32),
            pltpu.VMEM((SUB,), jnp.int32),
            pltpu.VMEM((TAILSZ,), jnp.int32),
            pltpu.VMEM((SUB,), jnp.float32),
            pltpu.SemaphoreType.DMA,
            pltpu.VMEM_SHARED((N,), jnp.float32),
        ],
    )(ei, w)
    return norm


def _prop_body(ei, nrm, v, t0, out, s128, s80, d128, d80, nb, gbuf, sem,
               acc_sp, *, first):
    cidx = lax.axis_index("c")
    sid = lax.axis_index("s")
    off_n = sid * RSTEP
    voff = cidx * N

    _zero_rows(gbuf, ECH)

    def z_body(j, _):
        pltpu.sync_copy(gbuf.at[pl.ds(0, ECH)],
                        acc_sp.at[pl.ds(off_n + j * ECH, ECH)])
        return 0

    lax.fori_loop(0, ROWS // ECH, z_body, 0)
    plsc.subcore_barrier()

    def s_body(oc, _):
        base = sid * EPT + oc * OCH
        pltpu.sync_copy(nrm.at[pl.ds(cidx * E + base, OCH)],
                        nb.at[pl.ds(0, OCH)])
        for j in range(NSUB + 1):
            sz = SUB if j < NSUB else TAILSZ
            sbuf = s128 if j < NSUB else s80
            dbuf = d128 if j < NSUB else d80
            pltpu.sync_copy(ei.at[pl.ds((1 - cidx) * E + base + j * SUB, sz)],
                            sbuf)
            pltpu.sync_copy(ei.at[pl.ds(cidx * E + base + j * SUB, sz)], dbuf)

            def a_body(g, _, sbuf=sbuf):
                sbuf[pl.ds(g * L, L)] = sbuf[pl.ds(g * L, L)] + voff
                return 0

            lax.fori_loop(0, sz // L, a_body, 0)
            pltpu.async_copy(v.at[sbuf], gbuf.at[pl.ds(0, sz)], sem).wait()

            def r_body(r, _, j=j):
                s = nb[pl.ds(j * SUB + r, L)][0]
                for q in range(D // L):
                    gbuf[r, pl.ds(q * L, L)] = gbuf[r, pl.ds(q * L, L)] * s
                return 0

            lax.fori_loop(0, sz, r_body, 0)
            pltpu.sync_copy(gbuf.at[pl.ds(0, sz)], acc_sp.at[dbuf], add=True)
        return 0

    lax.fori_loop(0, NOCH, s_body, 0)
    plsc.subcore_barrier()

    if first:
        def f_body(j, _):
            ro = off_n + j * ECH
            pltpu.sync_copy(acc_sp.at[pl.ds(ro, ECH)], gbuf.at[pl.ds(0, ECH)])
            pltpu.sync_copy(gbuf.at[pl.ds(0, ECH)],
                            out.at[pl.ds(voff + ro, ECH)])
            return 0

        lax.fori_loop(0, ROWS // ECH, f_body, 0)
    else:
        def e_body(j, _):
            ro = off_n + j * ECH
            pltpu.sync_copy(acc_sp.at[pl.ds(ro, ECH)], gbuf.at[pl.ds(0, ECH)])
            pltpu.sync_copy(t0.at[pl.ds(voff + ro, ECH)],
                            gbuf.at[pl.ds(ECH, ECH)])

            def c_body(i, _):
                r = i // (D // L)
                c = (i % (D // L)) * L
                a = gbuf[r, pl.ds(c, L)]
                t = gbuf[ECH + r, pl.ds(c, L)]
                gbuf[2 * ECH + r, pl.ds(c, L)] = a + a - t
                return 0

            lax.fori_loop(0, ECH * (D // L), c_body, 0)
            pltpu.sync_copy(gbuf.at[pl.ds(2 * ECH, ECH)],
                            out.at[pl.ds(voff + ro, ECH)])
            return 0

        lax.fori_loop(0, ROWS // ECH, e_body, 0)


def _prop_call(ei, nrm, v, t0, first):
    with compute_on("tpu_sparsecore"):
        return _prop_call_inner(ei, nrm, v, t0, first)


def _prop_call_inner(ei, nrm, v, t0, first):
    return pl.kernel(
        functools.partial(_prop_body, first=first),
        out_type=jax.ShapeDtypeStruct((NC * N, D), jnp.float32),
        mesh=_MESH,
        scratch_types=[
            pltpu.VMEM((SUB,), jnp.int32),
            pltpu.VMEM((TAILSZ,), jnp.int32),
            pltpu.VMEM((SUB,), jnp.int32),
            pltpu.VMEM((TAILSZ,), jnp.int32),
            pltpu.VMEM((OCH + L,), jnp.float32),
            pltpu.VMEM((3 * ECH, D), jnp.float32),
            pltpu.SemaphoreType.DMA,
            pltpu.VMEM_SHARED((N, D), jnp.float32),
        ],
    )(ei, nrm, v, t0)


_BLK = 1000


def _acc_body(t0_ref, t1_ref, w_ref, g_ref, o_ref):
    o_ref[...] = (g_ref[...]
                  + jnp.dot(t0_ref[...], w_ref[0],
                            preferred_element_type=jnp.float32)
                  + jnp.dot(t1_ref[...], w_ref[1],
                            preferred_element_type=jnp.float32))


def _acc(t, w2, g):
    nb = N // _BLK
    return pl.pallas_call(
        _acc_body,
        grid=(nb,),
        in_specs=[
            pl.BlockSpec((_BLK, D), lambda i: (i, 0)),
            pl.BlockSpec((_BLK, D), lambda i, nb=nb: (i + nb, 0)),
            pl.BlockSpec((2, D, 2 * HID), lambda i: (0, 0, 0)),
            pl.BlockSpec((_BLK, 2 * HID), lambda i: (i, 0)),
        ],
        out_specs=pl.BlockSpec((_BLK, 2 * HID), lambda i: (i, 0)),
        out_shape=jax.ShapeDtypeStruct((N, 2 * HID), jnp.float32),
    )(t, t, w2, g)


def _finish_body(g_ref, bzh_ref, wl_ref, bl_ref, o_ref):
    g = g_ref[...] + bzh_ref[...]
    z = jax.nn.sigmoid(g[:, :HID])
    ht = jnp.tanh(g[:, HID:])
    h = jax.nn.relu((1.0 - z) * ht)
    o_ref[...] = jnp.dot(h, wl_ref[...],
                         preferred_element_type=jnp.float32) + bl_ref[...]


def _finish(g, bzh, wl_pad, bl_pad):
    return pl.pallas_call(
        _finish_body,
        grid=(N // _BLK,),
        in_specs=[
            pl.BlockSpec((_BLK, 2 * HID), lambda i: (i, 0)),
            pl.BlockSpec((1, 2 * HID), lambda i: (0, 0)),
            pl.BlockSpec((HID, 128), lambda i: (0, 0)),
            pl.BlockSpec((1, 128), lambda i: (0, 0)),
        ],
        out_specs=pl.BlockSpec((_BLK, 128), lambda i: (i, 0)),
        out_shape=jax.ShapeDtypeStruct((N, 128), jnp.float32),
    )(g, bzh, wl_pad, bl_pad)


def kernel(x, edge_index, edge_weight, W_z, b_z, W_r, b_r, W_h, b_h, W_lin, b_lin):
    ei = edge_index.astype(jnp.int32).reshape(2 * E)
    w = edge_weight.astype(jnp.float32)

    nrm = _norm_call(ei, w)

    wz = W_z[:, :, :D, :]
    wh = W_h[:, :, :D, :]
    w_cat = jnp.concatenate([wz, wh], axis=-1)
    w_all = w_cat.transpose(1, 0, 2, 3)

    xx = jnp.concatenate([x, x], axis=0)
    g = _acc(xx, w_all[0], jnp.zeros((N, 2 * HID), jnp.float32))
    t1 = _prop_call(ei, nrm, xx, xx, True)
    g = _acc(t1, w_all[1], g)
    prev, cur = xx, t1
    for k in range(2, K):
        nxt = _prop_call(ei, nrm, cur, prev, False)
        g = _acc(nxt, w_all[k], g)
        prev, cur = cur, nxt

    bzh = jnp.concatenate([b_z, b_h]).reshape(1, 2 * HID)
    wl_pad = jnp.zeros((HID, 128), jnp.float32).at[:, :PRE_LEN].set(W_lin)
    bl_pad = jnp.zeros((1, 128), jnp.float32).at[0, :PRE_LEN].set(b_lin)

    out = _finish(g, bzh, wl_pad, bl_pad)
    return out[:, :PRE_LEN]


# speedup vs baseline: 10.7608x; 1.7524x over previous
import functools

import jax
import jax.numpy as jnp
from jax import lax
from jax.experimental.compute_on import compute_on
from jax.experimental import pallas as pl
from jax.experimental.pallas import tpu as pltpu
from jax.experimental.pallas import tpu_sc as plsc

N = 10000
E = 320000
D = 128
HID = 64
K = 16
PRE_LEN = 4

NC = 2
NS = 16
L = 16

EPT = E // NS
OCH = 2000
NOCH = EPT // OCH
SUB = 128
NSUB = 15
TAILSZ = OCH - NSUB * SUB
ROWS = 640
RSTEP = 624
ECH = 64
EROWS = E // SUB
O_SUB = 12
OCHE = O_SUB * SUB
ROWS_T = 156
NOUT = ROWS_T // O_SUB
XROW0 = NS * ROWS_T
X_SUB = EROWS - XROW0

_MESH = plsc.VectorSubcoreMesh(core_axis_name="c", subcore_axis_name="s",
                               num_cores=NC, num_subcores=NS)


def _zero_rows(buf, nrows):
    z = jnp.zeros((L,), jnp.float32)

    def body(i, _):
        r = i // (D // L)
        c = (i % (D // L)) * L
        buf[r, pl.ds(c, L)] = z
        return 0

    lax.fori_loop(0, nrows * (D // L), body, 0)


def _norm_body(ei, w, norm_out, deg_out, zb, wb, nb, d128, d80, dgb, sem,
               deg_sp):
    cidx = lax.axis_index("c")
    sid = lax.axis_index("s")
    off_n = sid * RSTEP

    def zb_body(g, _):
        zb[pl.ds(g * L, L)] = jnp.zeros((L,), jnp.float32)
        return 0

    lax.fori_loop(0, ROWS // L, zb_body, 0)
    pltpu.sync_copy(zb.at[pl.ds(0, ROWS)], deg_sp.at[pl.ds(off_n, ROWS)])
    plsc.subcore_barrier()

    def ch_body(oc, _):
        base = sid * EPT + oc * OCH
        pltpu.sync_copy(w.at[pl.ds(base, OCH)], wb)
        for j in range(NSUB + 1):
            sz = SUB if j < NSUB else TAILSZ
            dbuf = d128 if j < NSUB else d80
            pltpu.sync_copy(ei.at[pl.ds(cidx * E + base + j * SUB, sz)],
                            dbuf)
            pltpu.sync_copy(wb.at[pl.ds(j * SUB, sz)], deg_sp.at[dbuf],
                            add=True)
        return 0

    lax.fori_loop(0, NOCH, ch_body, 0)
    plsc.subcore_barrier()

    pltpu.sync_copy(deg_sp.at[pl.ds(off_n, ROWS)], zb.at[pl.ds(0, ROWS)])
    pltpu.sync_copy(zb.at[pl.ds(0, ROWS)],
                    deg_out.at[pl.ds(cidx * N + off_n, ROWS)])
    plsc.subcore_barrier()

    def nch_body(oc, _):
        base = sid * EPT + oc * OCH
        pltpu.sync_copy(w.at[pl.ds(base, OCH)], wb)
        for j in range(NSUB + 1):
            sz = SUB if j < NSUB else TAILSZ
            dbuf = d128 if j < NSUB else d80
            pltpu.sync_copy(ei.at[pl.ds(cidx * E + base + j * SUB, sz)],
                            dbuf)

            def adj_body(g, _, dbuf=dbuf):
                dbuf[pl.ds(g * L, L)] = dbuf[pl.ds(g * L, L)] + cidx * N
                return 0

            lax.fori_loop(0, sz // L, adj_body, 0)
            pltpu.async_copy(deg_out.at[dbuf], dgb.at[pl.ds(0, sz)],
                             sem).wait()

            def g_body(g, _, j=j):
                wv = wb[pl.ds(j * SUB + g * L, L)]
                dv = dgb[pl.ds(g * L, L)]
                nb[pl.ds(j * SUB + g * L, L)] = wv / dv
                return 0

            lax.fori_loop(0, sz // L, g_body, 0)
        pltpu.sync_copy(nb, norm_out.at[pl.ds(cidx * E + base, OCH)])
        return 0

    lax.fori_loop(0, NOCH, nch_body, 0)


def _norm_call(ei, w):
    with compute_on("tpu_sparsecore"):
        return _norm_call_inner(ei, w)


def _norm_call_inner(ei, w):
    norm, _deg = pl.kernel(
        _norm_body,
        out_type=(jax.ShapeDtypeStruct((NC * E,), jnp.float32),
                  jax.ShapeDtypeStruct((NC * N,), jnp.float32)),
        mesh=_MESH,
        scratch_types=[
            pltpu.VMEM((OCH,), jnp.float32),
            pltpu.VMEM((OCH,), jnp.float32),
            pltpu.VMEM((OCH,), jnp.float32),
            pltpu.VMEM((SUB,), jnp.int32),
            pltpu.VMEM((TAILSZ,), jnp.int32),
            pltpu.VMEM((SUB,), jnp.float32),
            pltpu.SemaphoreType.DMA,
            pltpu.VMEM_SHARED((N,), jnp.float32),
        ],
    )(ei, w)
    return norm


def _prop_body(ei, nrm, eid2, v, t0, out, sb, nb, d2, gA, gB, sem_m, sg0,
               sg1, ss0, ss1, acc_sp, *, first):
    cidx = lax.axis_index("c")
    sid = lax.axis_index("s")
    off_n = sid * RSTEP
    voff = cidx * N

    _zero_rows(gA, ECH)

    def z_body(j, _):
        pltpu.sync_copy(gA.at[pl.ds(0, ECH)],
                        acc_sp.at[pl.ds(off_n + j * ECH, ECH)])
        return 0

    lax.fori_loop(0, ROWS // ECH, z_body, 0)
    plsc.subcore_barrier()

    def scale(buf, nbase):
        def r_body(r, _):
            s = nb[pl.ds(nbase + r, L)][0]
            for q in range(D // L):
                buf[r, pl.ds(q * L, L)] = buf[r, pl.ds(q * L, L)] * s
            return 0

        lax.fori_loop(0, SUB, r_body, 0)

    def run_outer(row_base, nsub, bufs_sems):
        gA_, gB_, sg0_, sg1_, ss0_, ss1_ = bufs_sems
        ebase = row_base * SUB
        m1 = pltpu.async_copy(nrm.at[pl.ds(cidx * E + ebase, nsub * SUB)],
                              nb.at[pl.ds(0, nsub * SUB)], sem_m)
        m2 = pltpu.async_copy(ei.at[pl.ds((1 - cidx) * E + ebase,
                                          nsub * SUB)],
                              sb.at[pl.ds(0, nsub * SUB)], sem_m)
        gbase = cidx * EROWS + row_base
        base8 = (gbase // 8) * 8
        roff = gbase - base8
        nrows = ((nsub + 15) // 8) * 8
        m3 = pltpu.async_copy(eid2.at[pl.ds(base8, nrows)],
                              d2.at[pl.ds(0, nrows)], sem_m)
        m1.wait()
        m2.wait()
        m3.wait()

        def a_body(g, _):
            sb[pl.ds(g * L, L)] = sb[pl.ds(g * L, L)] + voff
            return 0

        lax.fori_loop(0, nsub * SUB // L, a_body, 0)

        gd = [None] * nsub
        sd = [None] * nsub
        gd[0] = pltpu.async_copy(v.at[sb.at[pl.ds(0, SUB)]], gA_, sg0_)
        for j in range(nsub):
            cur = gA_ if j % 2 == 0 else gB_
            gd[j].wait()
            if j + 1 < nsub:
                if j - 1 >= 0:
                    sd[j - 1].wait()
                nxt = gB_ if j % 2 == 0 else gA_
                sgn = sg1_ if j % 2 == 0 else sg0_
                gd[j + 1] = pltpu.async_copy(
                    v.at[sb.at[pl.ds((j + 1) * SUB, SUB)]], nxt, sgn)
            scale(cur, j * SUB)
            ssem = ss0_ if j % 2 == 0 else ss1_
            sd[j] = pltpu.async_copy(cur, acc_sp.at[d2.at[roff + j]], ssem,
                                     add=True)
        if nsub >= 2:
            sd[nsub - 2].wait()
        sd[nsub - 1].wait()

    bufs_sems = (gA, gB, sg0, sg1, ss0, ss1)

    def o_body(oc, _):
        run_outer(sid * ROWS_T + oc * O_SUB, O_SUB, bufs_sems)
        return 0

    lax.fori_loop(0, NOUT, o_body, 0)

    @pl.when(sid == NS - 1)
    def _():
        run_outer(XROW0, X_SUB, bufs_sems)

    plsc.subcore_barrier()

    if first:
        def f_body(j, _):
            ro = off_n + j * 2 * ECH
            pltpu.sync_copy(acc_sp.at[pl.ds(ro, 2 * ECH)], gA)
            pltpu.sync_copy(gA, out.at[pl.ds(voff + ro, 2 * ECH)])
            return 0

        lax.fori_loop(0, ROWS // (2 * ECH), f_body, 0)
    else:
        def e_body(j, _):
            ro = off_n + j * ECH
            pltpu.sync_copy(acc_sp.at[pl.ds(ro, ECH)], gA.at[pl.ds(0, ECH)])
            pltpu.sync_copy(t0.at[pl.ds(voff + ro, ECH)],
                            gA.at[pl.ds(ECH, ECH)])

            def c_body(i, _):
                r = i // (D // L)
                c = (i % (D // L)) * L
                a = gA[r, pl.ds(c, L)]
                t = gA[ECH + r, pl.ds(c, L)]
                gB[r, pl.ds(c, L)] = a + a - t
                return 0

            lax.fori_loop(0, ECH * (D // L), c_body, 0)
            pltpu.sync_copy(gB.at[pl.ds(0, ECH)],
                            out.at[pl.ds(voff + ro, ECH)])
            return 0

        lax.fori_loop(0, ROWS // ECH, e_body, 0)


def _prop_call(ei, nrm, eid2, v, t0, first):
    return pl.kernel(
        functools.partial(_prop_body, first=first),
        out_type=jax.ShapeDtypeStruct((NC * N, D), jnp.float32),
        mesh=_MESH,
        scratch_types=[
            pltpu.VMEM((OCHE,), jnp.int32),
            pltpu.VMEM((OCHE + L,), jnp.float32),
            pltpu.VMEM((O_SUB + 12, SUB), jnp.int32),
            pltpu.VMEM((SUB, D), jnp.float32),
            pltpu.VMEM((SUB, D), jnp.float32),
            pltpu.SemaphoreType.DMA,
            pltpu.SemaphoreType.DMA,
            pltpu.SemaphoreType.DMA,
            pltpu.SemaphoreType.DMA,
            pltpu.SemaphoreType.DMA,
            pltpu.VMEM_SHARED((N, D), jnp.float32),
        ],
    )(ei, nrm, eid2, v, t0)


_BLK = 1000


def _acc_body(t0_ref, t1_ref, w_ref, g_ref, o_ref):
    o_ref[...] = (g_ref[...]
                  + jnp.dot(t0_ref[...], w_ref[0],
                            preferred_element_type=jnp.float32)
                  + jnp.dot(t1_ref[...], w_ref[1],
                            preferred_element_type=jnp.float32))


def _acc(t, w2, g):
    nb = N // _BLK
    return pl.pallas_call(
        _acc_body,
        grid=(nb,),
        in_specs=[
            pl.BlockSpec((_BLK, D), lambda i: (i, 0)),
            pl.BlockSpec((_BLK, D), lambda i, nb=nb: (i + nb, 0)),
            pl.BlockSpec((2, D, 2 * HID), lambda i: (0, 0, 0)),
            pl.BlockSpec((_BLK, 2 * HID), lambda i: (i, 0)),
        ],
        out_specs=pl.BlockSpec((_BLK, 2 * HID), lambda i: (i, 0)),
        out_shape=jax.ShapeDtypeStruct((N, 2 * HID), jnp.float32),
    )(t, t, w2, g)


def _finish_body(g_ref, bzh_ref, wl_ref, bl_ref, o_ref):
    g = g_ref[...] + bzh_ref[...]
    z = jax.nn.sigmoid(g[:, :HID])
    ht = jnp.tanh(g[:, HID:])
    h = jax.nn.relu((1.0 - z) * ht)
    o_ref[...] = jnp.dot(h, wl_ref[...],
                         preferred_element_type=jnp.float32) + bl_ref[...]


def _finish(g, bzh, wl_pad, bl_pad):
    return pl.pallas_call(
        _finish_body,
        grid=(N // _BLK,),
        in_specs=[
            pl.BlockSpec((_BLK, 2 * HID), lambda i: (i, 0)),
            pl.BlockSpec((1, 2 * HID), lambda i: (0, 0)),
            pl.BlockSpec((HID, 128), lambda i: (0, 0)),
            pl.BlockSpec((1, 128), lambda i: (0, 0)),
        ],
        out_specs=pl.BlockSpec((_BLK, 128), lambda i: (i, 0)),
        out_shape=jax.ShapeDtypeStruct((N, 128), jnp.float32),
    )(g, bzh, wl_pad, bl_pad)


def kernel(x, edge_index, edge_weight, W_z, b_z, W_r, b_r, W_h, b_h, W_lin, b_lin):
    ei = edge_index.astype(jnp.int32).reshape(2 * E)
    w = edge_weight.astype(jnp.float32)

    nrm = _norm_call(ei, w)

    wz = W_z[:, :, :D, :]
    wh = W_h[:, :, :D, :]
    w_cat = jnp.concatenate([wz, wh], axis=-1)
    w_all = w_cat.transpose(1, 0, 2, 3)

    eid2 = jnp.concatenate(
        [ei.reshape(2 * EROWS, SUB), jnp.zeros((24, SUB), jnp.int32)])

    xx = jnp.concatenate([x, x], axis=0)
    g = _acc(xx, w_all[0], jnp.zeros((N, 2 * HID), jnp.float32))
    t1 = _prop_call(ei, nrm, eid2, xx, xx, True)
    g = _acc(t1, w_all[1], g)
    prev, cur = xx, t1
    for k in range(2, K):
        nxt = _prop_call(ei, nrm, eid2, cur, prev, False)
        g = _acc(nxt, w_all[k], g)
        prev, cur = cur, nxt

    bzh = jnp.concatenate([b_z, b_h]).reshape(1, 2 * HID)
    wl_pad = jnp.zeros((HID, 128), jnp.float32).at[:, :PRE_LEN].set(W_lin)
    bl_pad = jnp.zeros((1, 128), jnp.float32).at[0, :PRE_LEN].set(b_lin)

    out = _finish(g, bzh, wl_pad, bl_pad)
    return out[:, :PRE_LEN]


# speedup vs baseline: 12.8871x; 1.1976x over previous
import functools

import jax
import jax.numpy as jnp
from jax import lax
from jax.experimental.compute_on import compute_on
from jax.experimental import pallas as pl
from jax.experimental.pallas import tpu as pltpu
from jax.experimental.pallas import tpu_sc as plsc

N = 10000
E = 320000
D = 128
HID = 64
K = 16
PRE_LEN = 4

NC = 2
NS = 16
L = 16

EPT = E // NS
OCH = 2000
NOCH = EPT // OCH
SUB = 128
NSUB = 15
TAILSZ = OCH - NSUB * SUB
ROWS = 640
RSTEP = 624
ECH = 64
EROWS = E // SUB
O_SUB = 12
OCHE = O_SUB * SUB
ROWS_T = 156
NOUT = ROWS_T // O_SUB
XROW0 = NS * ROWS_T
X_SUB = EROWS - XROW0

_MESH = plsc.VectorSubcoreMesh(core_axis_name="c", subcore_axis_name="s",
                               num_cores=NC, num_subcores=NS)


def _zero_rows(buf, nrows):
    z = jnp.zeros((L,), jnp.float32)

    def body(i, _):
        r = i // (D // L)
        c = (i % (D // L)) * L
        buf[r, pl.ds(c, L)] = z
        return 0

    lax.fori_loop(0, nrows * (D // L), body, 0)


def _norm_body(ei, w, norm_out, deg_out, zb, wb, nb, d128, d80, dgb, sem,
               deg_sp):
    cidx = lax.axis_index("c")
    sid = lax.axis_index("s")
    off_n = sid * RSTEP

    def zb_body(g, _):
        zb[pl.ds(g * L, L)] = jnp.zeros((L,), jnp.float32)
        return 0

    lax.fori_loop(0, ROWS // L, zb_body, 0)
    pltpu.sync_copy(zb.at[pl.ds(0, ROWS)], deg_sp.at[pl.ds(off_n, ROWS)])
    plsc.subcore_barrier()

    def ch_body(oc, _):
        base = sid * EPT + oc * OCH
        pltpu.sync_copy(w.at[pl.ds(base, OCH)], wb)
        for j in range(NSUB + 1):
            sz = SUB if j < NSUB else TAILSZ
            dbuf = d128 if j < NSUB else d80
            pltpu.sync_copy(ei.at[pl.ds(cidx * E + base + j * SUB, sz)],
                            dbuf)
            pltpu.sync_copy(wb.at[pl.ds(j * SUB, sz)], deg_sp.at[dbuf],
                            add=True)
        return 0

    lax.fori_loop(0, NOCH, ch_body, 0)
    plsc.subcore_barrier()

    pltpu.sync_copy(deg_sp.at[pl.ds(off_n, ROWS)], zb.at[pl.ds(0, ROWS)])
    pltpu.sync_copy(zb.at[pl.ds(0, ROWS)],
                    deg_out.at[pl.ds(cidx * N + off_n, ROWS)])
    plsc.subcore_barrier()

    def nch_body(oc, _):
        base = sid * EPT + oc * OCH
        pltpu.sync_copy(w.at[pl.ds(base, OCH)], wb)
        for j in range(NSUB + 1):
            sz = SUB if j < NSUB else TAILSZ
            dbuf = d128 if j < NSUB else d80
            pltpu.sync_copy(ei.at[pl.ds(cidx * E + base + j * SUB, sz)],
                            dbuf)

            def adj_body(g, _, dbuf=dbuf):
                dbuf[pl.ds(g * L, L)] = dbuf[pl.ds(g * L, L)] + cidx * N
                return 0

            lax.fori_loop(0, sz // L, adj_body, 0)
            pltpu.async_copy(deg_out.at[dbuf], dgb.at[pl.ds(0, sz)],
                             sem).wait()

            def g_body(g, _, j=j):
                wv = wb[pl.ds(j * SUB + g * L, L)]
                dv = dgb[pl.ds(g * L, L)]
                nb[pl.ds(j * SUB + g * L, L)] = wv / dv
                return 0

            lax.fori_loop(0, sz // L, g_body, 0)
        pltpu.sync_copy(nb, norm_out.at[pl.ds(cidx * E + base, OCH)])
        return 0

    lax.fori_loop(0, NOCH, nch_body, 0)


def _norm_call(ei, w):
    with compute_on("tpu_sparsecore"):
        return _norm_call_inner(ei, w)


def _norm_call_inner(ei, w):
    norm, _deg = pl.kernel(
        _norm_body,
        out_type=(jax.ShapeDtypeStruct((NC * E,), jnp.float32),
                  jax.ShapeDtypeStruct((NC * N,), jnp.float32)),
        mesh=_MESH,
        scratch_types=[
            pltpu.VMEM((OCH,), jnp.float32),
            pltpu.VMEM((OCH,), jnp.float32),
            pltpu.VMEM((OCH,), jnp.float32),
            pltpu.VMEM((SUB,), jnp.int32),
            pltpu.VMEM((TAILSZ,), jnp.int32),
            pltpu.VMEM((SUB,), jnp.float32),
            pltpu.SemaphoreType.DMA,
            pltpu.VMEM_SHARED((N,), jnp.float32),
        ],
    )(ei, w)
    return norm


def _prop_body(ei, nrm, eid2, v, t0, out, sb, nb, d2, gA, gB, sem_m, sg0,
               sg1, ss0, ss1, acc_sp, *, first):
    cidx = lax.axis_index("c")
    sid = lax.axis_index("s")
    off_n = sid * RSTEP
    voff = cidx * N

    _zero_rows(gA, ECH)

    def z_body(j, _):
        pltpu.sync_copy(gA.at[pl.ds(0, ECH)],
                        acc_sp.at[pl.ds(off_n + j * ECH, ECH)])
        return 0

    lax.fori_loop(0, ROWS // ECH, z_body, 0)
    plsc.subcore_barrier()

    def scale(buf, nbase):
        @plsc.parallel_loop(0, SUB, 2, unroll=4)
        def r_body(r):
            for rr in range(2):
                s = nb[pl.ds(nbase + r + rr, L)][0]
                for q in range(D // L):
                    buf[r + rr, pl.ds(q * L, L)] = (
                        buf[r + rr, pl.ds(q * L, L)] * s)

    def run_outer(row_base, nsub, bufs_sems):
        gA_, gB_, sg0_, sg1_, ss0_, ss1_ = bufs_sems
        ebase = row_base * SUB
        m1 = pltpu.async_copy(nrm.at[pl.ds(cidx * E + ebase, nsub * SUB)],
                              nb.at[pl.ds(0, nsub * SUB)], sem_m)
        m2 = pltpu.async_copy(ei.at[pl.ds((1 - cidx) * E + ebase,
                                          nsub * SUB)],
                              sb.at[pl.ds(0, nsub * SUB)], sem_m)
        gbase = cidx * EROWS + row_base
        base8 = (gbase // 8) * 8
        roff = gbase - base8
        nrows = ((nsub + 15) // 8) * 8
        m3 = pltpu.async_copy(eid2.at[pl.ds(base8, nrows)],
                              d2.at[pl.ds(0, nrows)], sem_m)
        m1.wait()
        m2.wait()
        m3.wait()

        def a_body(g, _):
            sb[pl.ds(g * L, L)] = sb[pl.ds(g * L, L)] + voff
            return 0

        lax.fori_loop(0, nsub * SUB // L, a_body, 0)

        gd = [None] * nsub
        sd = [None] * nsub
        gd[0] = pltpu.async_copy(v.at[sb.at[pl.ds(0, SUB)]], gA_, sg0_)
        for j in range(nsub):
            cur = gA_ if j % 2 == 0 else gB_
            gd[j].wait()
            if j + 1 < nsub:
                if j - 1 >= 0:
                    sd[j - 1].wait()
                nxt = gB_ if j % 2 == 0 else gA_
                sgn = sg1_ if j % 2 == 0 else sg0_
                gd[j + 1] = pltpu.async_copy(
                    v.at[sb.at[pl.ds((j + 1) * SUB, SUB)]], nxt, sgn)
            scale(cur, j * SUB)
            ssem = ss0_ if j % 2 == 0 else ss1_
            sd[j] = pltpu.async_copy(cur, acc_sp.at[d2.at[roff + j]], ssem,
                                     add=True)
        if nsub >= 2:
            sd[nsub - 2].wait()
        sd[nsub - 1].wait()

    bufs_sems = (gA, gB, sg0, sg1, ss0, ss1)

    def o_body(oc, _):
        run_outer(sid * ROWS_T + oc * O_SUB, O_SUB, bufs_sems)
        return 0

    lax.fori_loop(0, NOUT, o_body, 0)

    @pl.when(sid == NS - 1)
    def _():
        run_outer(XROW0, X_SUB, bufs_sems)

    plsc.subcore_barrier()

    if first:
        def f_body(j, _):
            ro = off_n + j * 2 * ECH
            pltpu.sync_copy(acc_sp.at[pl.ds(ro, 2 * ECH)], gA)
            pltpu.sync_copy(gA, out.at[pl.ds(voff + ro, 2 * ECH)])
            return 0

        lax.fori_loop(0, ROWS // (2 * ECH), f_body, 0)
    else:
        def e_body(j, _):
            ro = off_n + j * ECH
            pltpu.sync_copy(acc_sp.at[pl.ds(ro, ECH)], gA.at[pl.ds(0, ECH)])
            pltpu.sync_copy(t0.at[pl.ds(voff + ro, ECH)],
                            gA.at[pl.ds(ECH, ECH)])

            def c_body(i, _):
                r = i // (D // L)
                c = (i % (D // L)) * L
                a = gA[r, pl.ds(c, L)]
                t = gA[ECH + r, pl.ds(c, L)]
                gB[r, pl.ds(c, L)] = a + a - t
                return 0

            lax.fori_loop(0, ECH * (D // L), c_body, 0)
            pltpu.sync_copy(gB.at[pl.ds(0, ECH)],
                            out.at[pl.ds(voff + ro, ECH)])
            return 0

        lax.fori_loop(0, ROWS // ECH, e_body, 0)


def _prop_call(ei, nrm, eid2, v, t0, first):
    return pl.kernel(
        functools.partial(_prop_body, first=first),
        out_type=jax.ShapeDtypeStruct((NC * N, D), jnp.float32),
        mesh=_MESH,
        scratch_types=[
            pltpu.VMEM((OCHE,), jnp.int32),
            pltpu.VMEM((OCHE + L,), jnp.float32),
            pltpu.VMEM((O_SUB + 12, SUB), jnp.int32),
            pltpu.VMEM((SUB, D), jnp.float32),
            pltpu.VMEM((SUB, D), jnp.float32),
            pltpu.SemaphoreType.DMA,
            pltpu.SemaphoreType.DMA,
            pltpu.SemaphoreType.DMA,
            pltpu.SemaphoreType.DMA,
            pltpu.SemaphoreType.DMA,
            pltpu.VMEM_SHARED((N, D), jnp.float32),
        ],
    )(ei, nrm, eid2, v, t0)


_BLK = 1000


def _acc_body(t0_ref, t1_ref, w_ref, g_ref, o_ref):
    o_ref[...] = (g_ref[...]
                  + jnp.dot(t0_ref[...], w_ref[0],
                            preferred_element_type=jnp.float32)
                  + jnp.dot(t1_ref[...], w_ref[1],
                            preferred_element_type=jnp.float32))


def _acc(t, w2, g):
    nb = N // _BLK
    return pl.pallas_call(
        _acc_body,
        grid=(nb,),
        in_specs=[
            pl.BlockSpec((_BLK, D), lambda i: (i, 0)),
            pl.BlockSpec((_BLK, D), lambda i, nb=nb: (i + nb, 0)),
            pl.BlockSpec((2, D, 2 * HID), lambda i: (0, 0, 0)),
            pl.BlockSpec((_BLK, 2 * HID), lambda i: (i, 0)),
        ],
        out_specs=pl.BlockSpec((_BLK, 2 * HID), lambda i: (i, 0)),
        out_shape=jax.ShapeDtypeStruct((N, 2 * HID), jnp.float32),
    )(t, t, w2, g)


def _finish_body(g_ref, bzh_ref, wl_ref, bl_ref, o_ref):
    g = g_ref[...] + bzh_ref[...]
    z = jax.nn.sigmoid(g[:, :HID])
    ht = jnp.tanh(g[:, HID:])
    h = jax.nn.relu((1.0 - z) * ht)
    o_ref[...] = jnp.dot(h, wl_ref[...],
                         preferred_element_type=jnp.float32) + bl_ref[...]


def _finish(g, bzh, wl_pad, bl_pad):
    return pl.pallas_call(
        _finish_body,
        grid=(N // _BLK,),
        in_specs=[
            pl.BlockSpec((_BLK, 2 * HID), lambda i: (i, 0)),
            pl.BlockSpec((1, 2 * HID), lambda i: (0, 0)),
            pl.BlockSpec((HID, 128), lambda i: (0, 0)),
            pl.BlockSpec((1, 128), lambda i: (0, 0)),
        ],
        out_specs=pl.BlockSpec((_BLK, 128), lambda i: (i, 0)),
        out_shape=jax.ShapeDtypeStruct((N, 128), jnp.float32),
    )(g, bzh, wl_pad, bl_pad)


def kernel(x, edge_index, edge_weight, W_z, b_z, W_r, b_r, W_h, b_h, W_lin, b_lin):
    ei = edge_index.astype(jnp.int32).reshape(2 * E)
    w = edge_weight.astype(jnp.float32)

    nrm = _norm_call(ei, w)

    wz = W_z[:, :, :D, :]
    wh = W_h[:, :, :D, :]
    w_cat = jnp.concatenate([wz, wh], axis=-1)
    w_all = w_cat.transpose(1, 0, 2, 3)

    eid2 = jnp.concatenate(
        [ei.reshape(2 * EROWS, SUB), jnp.zeros((24, SUB), jnp.int32)])

    xx = jnp.concatenate([x, x], axis=0)
    g = _acc(xx, w_all[0], jnp.zeros((N, 2 * HID), jnp.float32))
    t1 = _prop_call(ei, nrm, eid2, xx, xx, True)
    g = _acc(t1, w_all[1], g)
    prev, cur = xx, t1
    for k in range(2, K):
        nxt = _prop_call(ei, nrm, eid2, cur, prev, False)
        g = _acc(nxt, w_all[k], g)
        prev, cur = cur, nxt

    bzh = jnp.concatenate([b_z, b_h]).reshape(1, 2 * HID)
    wl_pad = jnp.zeros((HID, 128), jnp.float32).at[:, :PRE_LEN].set(W_lin)
    bl_pad = jnp.zeros((1, 128), jnp.float32).at[0, :PRE_LEN].set(b_lin)

    out = _finish(g, bzh, wl_pad, bl_pad)
    return out[:, :PRE_LEN]


# speedup vs baseline: 13.8096x; 1.0716x over previous
import functools

import jax
import jax.numpy as jnp
from jax import lax
from jax.experimental.compute_on import compute_on
from jax.experimental import pallas as pl
from jax.experimental.pallas import tpu as pltpu
from jax.experimental.pallas import tpu_sc as plsc

N = 10000
E = 320000
D = 128
HID = 64
K = 16
PRE_LEN = 4

NC = 2
NS = 16
L = 16

EPT = E // NS
OCH = 2000
NOCH = EPT // OCH
SUB = 128
NSUB = 15
TAILSZ = OCH - NSUB * SUB
ROWS = 640
RSTEP = 624
ECH = 64
EROWS = E // SUB
O_SUB = 12
OCHE = O_SUB * SUB
ROWS_T = 156
NOUT = ROWS_T // O_SUB
XROW0 = NS * ROWS_T
X_SUB = EROWS - XROW0

_MESH = plsc.VectorSubcoreMesh(core_axis_name="c", subcore_axis_name="s",
                               num_cores=NC, num_subcores=NS)


def _zero_rows(buf, nrows):
    z = jnp.zeros((L,), jnp.float32)

    def body(i, _):
        r = i // (D // L)
        c = (i % (D // L)) * L
        buf[r, pl.ds(c, L)] = z
        return 0

    lax.fori_loop(0, nrows * (D // L), body, 0)


def _norm_body(ei, w, norm_out, deg_out, zb, wb, nb, d128, d80, dgb, sem,
               deg_sp):
    cidx = lax.axis_index("c")
    sid = lax.axis_index("s")
    off_n = sid * RSTEP

    def zb_body(g, _):
        zb[pl.ds(g * L, L)] = jnp.zeros((L,), jnp.float32)
        return 0

    lax.fori_loop(0, ROWS // L, zb_body, 0)
    pltpu.sync_copy(zb.at[pl.ds(0, ROWS)], deg_sp.at[pl.ds(off_n, ROWS)])
    plsc.subcore_barrier()

    def ch_body(oc, _):
        base = sid * EPT + oc * OCH
        pltpu.sync_copy(w.at[pl.ds(base, OCH)], wb)
        for j in range(NSUB + 1):
            sz = SUB if j < NSUB else TAILSZ
            dbuf = d128 if j < NSUB else d80
            pltpu.sync_copy(ei.at[pl.ds(cidx * E + base + j * SUB, sz)],
                            dbuf)
            pltpu.sync_copy(wb.at[pl.ds(j * SUB, sz)], deg_sp.at[dbuf],
                            add=True)
        return 0

    lax.fori_loop(0, NOCH, ch_body, 0)
    plsc.subcore_barrier()

    pltpu.sync_copy(deg_sp.at[pl.ds(off_n, ROWS)], zb.at[pl.ds(0, ROWS)])
    pltpu.sync_copy(zb.at[pl.ds(0, ROWS)],
                    deg_out.at[pl.ds(cidx * N + off_n, ROWS)])
    plsc.subcore_barrier()

    def nch_body(oc, _):
        base = sid * EPT + oc * OCH
        pltpu.sync_copy(w.at[pl.ds(base, OCH)], wb)
        for j in range(NSUB + 1):
            sz = SUB if j < NSUB else TAILSZ
            dbuf = d128 if j < NSUB else d80
            pltpu.sync_copy(ei.at[pl.ds(cidx * E + base + j * SUB, sz)],
                            dbuf)

            def adj_body(g, _, dbuf=dbuf):
                dbuf[pl.ds(g * L, L)] = dbuf[pl.ds(g * L, L)] + cidx * N
                return 0

            lax.fori_loop(0, sz // L, adj_body, 0)
            pltpu.async_copy(deg_out.at[dbuf], dgb.at[pl.ds(0, sz)],
                             sem).wait()

            def g_body(g, _, j=j):
                wv = wb[pl.ds(j * SUB + g * L, L)]
                dv = dgb[pl.ds(g * L, L)]
                nb[pl.ds(j * SUB + g * L, L)] = wv / dv
                return 0

            lax.fori_loop(0, sz // L, g_body, 0)
        pltpu.sync_copy(nb, norm_out.at[pl.ds(cidx * E + base, OCH)])
        return 0

    lax.fori_loop(0, NOCH, nch_body, 0)


def _norm_call(ei, w):
    with compute_on("tpu_sparsecore"):
        return _norm_call_inner(ei, w)


def _norm_call_inner(ei, w):
    norm, _deg = pl.kernel(
        _norm_body,
        out_type=(jax.ShapeDtypeStruct((NC * E,), jnp.float32),
                  jax.ShapeDtypeStruct((NC * N,), jnp.float32)),
        mesh=_MESH,
        scratch_types=[
            pltpu.VMEM((OCH,), jnp.float32),
            pltpu.VMEM((OCH,), jnp.float32),
            pltpu.VMEM((OCH,), jnp.float32),
            pltpu.VMEM((SUB,), jnp.int32),
            pltpu.VMEM((TAILSZ,), jnp.int32),
            pltpu.VMEM((SUB,), jnp.float32),
            pltpu.SemaphoreType.DMA,
            pltpu.VMEM_SHARED((N,), jnp.float32),
        ],
    )(ei, w)
    return norm


def _prop_body(ei, nrm, eid2, v, t0, out, sb, nb, d2, gA, gB, sem_m, sg0,
               sg1, ss0, ss1, acc_sp, *, first):
    cidx = lax.axis_index("c")
    sid = lax.axis_index("s")
    off_n = sid * RSTEP
    voff = cidx * N

    _zero_rows(gA, ECH)

    def z_body(j, _):
        pltpu.sync_copy(gA.at[pl.ds(0, ECH)],
                        acc_sp.at[pl.ds(off_n + j * ECH, ECH)])
        return 0

    lax.fori_loop(0, ROWS // ECH, z_body, 0)
    plsc.subcore_barrier()

    def scale(buf, nbase):
        @plsc.parallel_loop(0, SUB, 2, unroll=4)
        def r_body(r):
            for rr in range(2):
                s = nb[pl.ds(nbase + r + rr, L)][0]
                for q in range(D // L):
                    buf[r + rr, pl.ds(q * L, L)] = (
                        buf[r + rr, pl.ds(q * L, L)] * s)

    def run_outer(row_base, nsub, bufs_sems):
        gA_, gB_, sg0_, sg1_, ss0_, ss1_ = bufs_sems
        ebase = row_base * SUB
        m1 = pltpu.async_copy(nrm.at[pl.ds(cidx * E + ebase, nsub * SUB)],
                              nb.at[pl.ds(0, nsub * SUB)], sem_m)
        m2 = pltpu.async_copy(ei.at[pl.ds((1 - cidx) * E + ebase,
                                          nsub * SUB)],
                              sb.at[pl.ds(0, nsub * SUB)], sem_m)
        gbase = cidx * EROWS + row_base
        base8 = (gbase // 8) * 8
        roff = gbase - base8
        nrows = ((nsub + 15) // 8) * 8
        m3 = pltpu.async_copy(eid2.at[pl.ds(base8, nrows)],
                              d2.at[pl.ds(0, nrows)], sem_m)
        m1.wait()
        m2.wait()
        m3.wait()

        def a_body(g, _):
            sb[pl.ds(g * L, L)] = sb[pl.ds(g * L, L)] + voff
            return 0

        lax.fori_loop(0, nsub * SUB // L, a_body, 0)

        gd = [None] * nsub
        sd = [None] * nsub
        gd[0] = pltpu.async_copy(v.at[sb.at[pl.ds(0, SUB)]], gA_, sg0_)
        for j in range(nsub):
            cur = gA_ if j % 2 == 0 else gB_
            gd[j].wait()
            if j + 1 < nsub:
                if j - 1 >= 0:
                    sd[j - 1].wait()
                nxt = gB_ if j % 2 == 0 else gA_
                sgn = sg1_ if j % 2 == 0 else sg0_
                gd[j + 1] = pltpu.async_copy(
                    v.at[sb.at[pl.ds((j + 1) * SUB, SUB)]], nxt, sgn)
            scale(cur, j * SUB)
            ssem = ss0_ if j % 2 == 0 else ss1_
            sd[j] = pltpu.async_copy(cur, acc_sp.at[d2.at[roff + j]], ssem,
                                     add=True)
        if nsub >= 2:
            sd[nsub - 2].wait()
        sd[nsub - 1].wait()

    bufs_sems = (gA, gB, sg0, sg1, ss0, ss1)

    def o_body(oc, _):
        run_outer(sid * ROWS_T + oc * O_SUB, O_SUB, bufs_sems)
        return 0

    lax.fori_loop(0, NOUT, o_body, 0)

    @pl.when(sid == NS - 1)
    def _():
        run_outer(XROW0, X_SUB, bufs_sems)

    plsc.subcore_barrier()

    if first:
        def f_body(j, _):
            ro = off_n + j * 2 * ECH
            pltpu.sync_copy(acc_sp.at[pl.ds(ro, 2 * ECH)], gA)
            pltpu.sync_copy(gA, out.at[pl.ds(voff + ro, 2 * ECH)])
            return 0

        lax.fori_loop(0, ROWS // (2 * ECH), f_body, 0)
    else:
        def e_body(j, _):
            ro = off_n + j * ECH
            pltpu.sync_copy(acc_sp.at[pl.ds(ro, ECH)], gA.at[pl.ds(0, ECH)])
            pltpu.sync_copy(t0.at[pl.ds(voff + ro, ECH)],
                            gA.at[pl.ds(ECH, ECH)])

            @plsc.parallel_loop(0, ECH * (D // L), 2, unroll=4)
            def c_body(i):
                for u in range(2):
                    r = (i + u) // (D // L)
                    c = ((i + u) % (D // L)) * L
                    a = gA[r, pl.ds(c, L)]
                    t = gA[ECH + r, pl.ds(c, L)]
                    gB[r, pl.ds(c, L)] = a + a - t

            pltpu.sync_copy(gB.at[pl.ds(0, ECH)],
                            out.at[pl.ds(voff + ro, ECH)])
            return 0

        lax.fori_loop(0, ROWS // ECH, e_body, 0)


def _prop_call(ei, nrm, eid2, v, t0, first):
    return pl.kernel(
        functools.partial(_prop_body, first=first),
        out_type=jax.ShapeDtypeStruct((NC * N, D), jnp.float32),
        mesh=_MESH,
        scratch_types=[
            pltpu.VMEM((OCHE,), jnp.int32),
            pltpu.VMEM((OCHE + L,), jnp.float32),
            pltpu.VMEM((O_SUB + 12, SUB), jnp.int32),
            pltpu.VMEM((SUB, D), jnp.float32),
            pltpu.VMEM((SUB, D), jnp.float32),
            pltpu.SemaphoreType.DMA,
            pltpu.SemaphoreType.DMA,
            pltpu.SemaphoreType.DMA,
            pltpu.SemaphoreType.DMA,
            pltpu.SemaphoreType.DMA,
            pltpu.VMEM_SHARED((N, D), jnp.float32),
        ],
    )(ei, nrm, eid2, v, t0)


_BLK = 1000


def _acc_body(t0_ref, t1_ref, w_ref, g_ref, o_ref):
    o_ref[...] = (g_ref[...]
                  + jnp.dot(t0_ref[...], w_ref[0],
                            preferred_element_type=jnp.float32)
                  + jnp.dot(t1_ref[...], w_ref[1],
                            preferred_element_type=jnp.float32))


def _acc(t, w2, g):
    nb = N // _BLK
    return pl.pallas_call(
        _acc_body,
        grid=(nb,),
        in_specs=[
            pl.BlockSpec((_BLK, D), lambda i: (i, 0)),
            pl.BlockSpec((_BLK, D), lambda i, nb=nb: (i + nb, 0)),
            pl.BlockSpec((2, D, 2 * HID), lambda i: (0, 0, 0)),
            pl.BlockSpec((_BLK, 2 * HID), lambda i: (i, 0)),
        ],
        out_specs=pl.BlockSpec((_BLK, 2 * HID), lambda i: (i, 0)),
        out_shape=jax.ShapeDtypeStruct((N, 2 * HID), jnp.float32),
    )(t, t, w2, g)


def _finish_body(g_ref, bzh_ref, wl_ref, bl_ref, o_ref):
    g = g_ref[...] + bzh_ref[...]
    z = jax.nn.sigmoid(g[:, :HID])
    ht = jnp.tanh(g[:, HID:])
    h = jax.nn.relu((1.0 - z) * ht)
    o_ref[...] = jnp.dot(h, wl_ref[...],
                         preferred_element_type=jnp.float32) + bl_ref[...]


def _finish(g, bzh, wl_pad, bl_pad):
    return pl.pallas_call(
        _finish_body,
        grid=(N // _BLK,),
        in_specs=[
            pl.BlockSpec((_BLK, 2 * HID), lambda i: (i, 0)),
            pl.BlockSpec((1, 2 * HID), lambda i: (0, 0)),
            pl.BlockSpec((HID, 128), lambda i: (0, 0)),
            pl.BlockSpec((1, 128), lambda i: (0, 0)),
        ],
        out_specs=pl.BlockSpec((_BLK, 128), lambda i: (i, 0)),
        out_shape=jax.ShapeDtypeStruct((N, 128), jnp.float32),
    )(g, bzh, wl_pad, bl_pad)


def kernel(x, edge_index, edge_weight, W_z, b_z, W_r, b_r, W_h, b_h, W_lin, b_lin):
    ei = edge_index.astype(jnp.int32).reshape(2 * E)
    w = edge_weight.astype(jnp.float32)

    nrm = _norm_call(ei, w)

    wz = W_z[:, :, :D, :]
    wh = W_h[:, :, :D, :]
    w_cat = jnp.concatenate([wz, wh], axis=-1)
    w_all = w_cat.transpose(1, 0, 2, 3)

    eid2 = jnp.concatenate(
        [ei.reshape(2 * EROWS, SUB), jnp.zeros((24, SUB), jnp.int32)])

    xx = jnp.concatenate([x, x], axis=0)
    g = _acc(xx, w_all[0], jnp.zeros((N, 2 * HID), jnp.float32))
    t1 = _prop_call(ei, nrm, eid2, xx, xx, True)
    g = _acc(t1, w_all[1], g)
    prev, cur = xx, t1
    for k in range(2, K):
        nxt = _prop_call(ei, nrm, eid2, cur, prev, False)
        g = _acc(nxt, w_all[k], g)
        prev, cur = cur, nxt

    bzh = jnp.concatenate([b_z, b_h]).reshape(1, 2 * HID)
    wl_pad = jnp.zeros((HID, 128), jnp.float32).at[:, :PRE_LEN].set(W_lin)
    bl_pad = jnp.zeros((1, 128), jnp.float32).at[0, :PRE_LEN].set(b_lin)

    out = _finish(g, bzh, wl_pad, bl_pad)
    return out[:, :PRE_LEN]


# speedup vs baseline: 13.8482x; 1.0028x over previous
import functools

import jax
import jax.numpy as jnp
from jax import lax
from jax.experimental.compute_on import compute_on
from jax.experimental import pallas as pl
from jax.experimental.pallas import tpu as pltpu
from jax.experimental.pallas import tpu_sc as plsc

N = 10000
E = 320000
D = 128
HID = 64
K = 16
PRE_LEN = 4

NC = 2
NS = 16
L = 16

EPT = E // NS
OCH = 2000
NOCH = EPT // OCH
SUB = 128
NSUB = 15
TAILSZ = OCH - NSUB * SUB
ROWS = 640
RSTEP = 624
ECH = 64
EROWS = E // SUB
O_SUB = 12
OCHE = O_SUB * SUB
ROWS_T = 156
NOUT = ROWS_T // O_SUB
XROW0 = NS * ROWS_T
X_SUB = EROWS - XROW0

_MESH = plsc.VectorSubcoreMesh(core_axis_name="c", subcore_axis_name="s",
                               num_cores=NC, num_subcores=NS)


def _zero_rows(buf, nrows):
    z = jnp.zeros((L,), jnp.float32)

    def body(i, _):
        r = i // (D // L)
        c = (i % (D // L)) * L
        buf[r, pl.ds(c, L)] = z
        return 0

    lax.fori_loop(0, nrows * (D // L), body, 0)


def _norm_body(ei, w, norm_out, deg_out, zb, wb, nb, d128, d80, dgb, sem,
               deg_sp):
    cidx = lax.axis_index("c")
    sid = lax.axis_index("s")
    off_n = sid * RSTEP

    def zb_body(g, _):
        zb[pl.ds(g * L, L)] = jnp.zeros((L,), jnp.float32)
        return 0

    lax.fori_loop(0, ROWS // L, zb_body, 0)
    pltpu.sync_copy(zb.at[pl.ds(0, ROWS)], deg_sp.at[pl.ds(off_n, ROWS)])
    plsc.subcore_barrier()

    def ch_body(oc, _):
        base = sid * EPT + oc * OCH
        pltpu.sync_copy(w.at[pl.ds(base, OCH)], wb)
        for j in range(NSUB + 1):
            sz = SUB if j < NSUB else TAILSZ
            dbuf = d128 if j < NSUB else d80
            pltpu.sync_copy(ei.at[pl.ds(cidx * E + base + j * SUB, sz)],
                            dbuf)
            pltpu.sync_copy(wb.at[pl.ds(j * SUB, sz)], deg_sp.at[dbuf],
                            add=True)
        return 0

    lax.fori_loop(0, NOCH, ch_body, 0)
    plsc.subcore_barrier()

    pltpu.sync_copy(deg_sp.at[pl.ds(off_n, ROWS)], zb.at[pl.ds(0, ROWS)])
    pltpu.sync_copy(zb.at[pl.ds(0, ROWS)],
                    deg_out.at[pl.ds(cidx * N + off_n, ROWS)])
    plsc.subcore_barrier()

    def nch_body(oc, _):
        base = sid * EPT + oc * OCH
        pltpu.sync_copy(w.at[pl.ds(base, OCH)], wb)
        for j in range(NSUB + 1):
            sz = SUB if j < NSUB else TAILSZ
            dbuf = d128 if j < NSUB else d80
            pltpu.sync_copy(ei.at[pl.ds(cidx * E + base + j * SUB, sz)],
                            dbuf)

            def adj_body(g, _, dbuf=dbuf):
                dbuf[pl.ds(g * L, L)] = dbuf[pl.ds(g * L, L)] + cidx * N
                return 0

            lax.fori_loop(0, sz // L, adj_body, 0)
            pltpu.async_copy(deg_out.at[dbuf], dgb.at[pl.ds(0, sz)],
                             sem).wait()

            def g_body(g, _, j=j):
                wv = wb[pl.ds(j * SUB + g * L, L)]
                dv = dgb[pl.ds(g * L, L)]
                nb[pl.ds(j * SUB + g * L, L)] = wv / dv
                return 0

            lax.fori_loop(0, sz // L, g_body, 0)
        pltpu.sync_copy(nb, norm_out.at[pl.ds(cidx * E + base, OCH)])
        return 0

    lax.fori_loop(0, NOCH, nch_body, 0)


def _norm_call(ei, w):
    with compute_on("tpu_sparsecore"):
        return _norm_call_inner(ei, w)


def _norm_call_inner(ei, w):
    norm, _deg = pl.kernel(
        _norm_body,
        out_type=(jax.ShapeDtypeStruct((NC * E,), jnp.float32),
                  jax.ShapeDtypeStruct((NC * N,), jnp.float32)),
        mesh=_MESH,
        scratch_types=[
            pltpu.VMEM((OCH,), jnp.float32),
            pltpu.VMEM((OCH,), jnp.float32),
            pltpu.VMEM((OCH,), jnp.float32),
            pltpu.VMEM((SUB,), jnp.int32),
            pltpu.VMEM((TAILSZ,), jnp.int32),
            pltpu.VMEM((SUB,), jnp.float32),
            pltpu.SemaphoreType.DMA,
            pltpu.VMEM_SHARED((N,), jnp.float32),
        ],
    )(ei, w)
    return norm


def _prop_body(ei, nrm, eid2, v, t0, out, sbA, sbB, nbA, nbB, d2A, d2B, gA,
               gB, sem_m, sg0, sg1, ss0, ss1, acc_sp, *, first):
    cidx = lax.axis_index("c")
    sid = lax.axis_index("s")
    off_n = sid * RSTEP
    voff = cidx * N

    _zero_rows(gA, ECH)
    zd = [pltpu.async_copy(gA.at[pl.ds(0, ECH)],
                           acc_sp.at[pl.ds(off_n + j * ECH, ECH)], sem_m)
          for j in range(ROWS // ECH)]
    for d in zd:
        d.wait()
    plsc.subcore_barrier()

    def scale(buf, nbase):
        @plsc.parallel_loop(0, SUB, 2, unroll=4)
        def r_body(r):
            for rr in range(2):
                s = nb[pl.ds(nbase + r + rr, L)][0]
                for q in range(D // L):
                    buf[r + rr, pl.ds(q * L, L)] = (
                        buf[r + rr, pl.ds(q * L, L)] * s)

    def meta_issue(row_base, nsub, p):
        ebase = row_base * SUB
        gbase = cidx * EROWS + row_base
        base8 = (gbase // 8) * 8
        nrows = ((nsub + 15) // 8) * 8
        nbp = nbA if p == 0 else nbB
        sbp = sbA if p == 0 else sbB
        d2p = d2A if p == 0 else d2B
        m1 = pltpu.async_copy(nrm.at[pl.ds(cidx * E + ebase, nsub * SUB)],
                              nbp.at[pl.ds(0, nsub * SUB)], sem_m)
        m2 = pltpu.async_copy(ei.at[pl.ds((1 - cidx) * E + ebase,
                                          nsub * SUB)],
                              sbp.at[pl.ds(0, nsub * SUB)], sem_m)
        m3 = pltpu.async_copy(eid2.at[pl.ds(base8, nrows)],
                              d2p.at[pl.ds(0, nrows)], sem_m)
        return (m1, m2, m3)

    def meta_wait(row_base, nsub, p):
        ebase = row_base * SUB
        gbase = cidx * EROWS + row_base
        base8 = (gbase // 8) * 8
        nrows = ((nsub + 15) // 8) * 8
        nbp = nbA if p == 0 else nbB
        sbp = sbA if p == 0 else sbB
        d2p = d2A if p == 0 else d2B
        pltpu.make_async_copy(nrm.at[pl.ds(cidx * E + ebase, nsub * SUB)],
                              nbp.at[pl.ds(0, nsub * SUB)], sem_m).wait()
        pltpu.make_async_copy(ei.at[pl.ds((1 - cidx) * E + ebase,
                                          nsub * SUB)],
                              sbp.at[pl.ds(0, nsub * SUB)], sem_m).wait()
        pltpu.make_async_copy(eid2.at[pl.ds(base8, nrows)],
                              d2p.at[pl.ds(0, nrows)], sem_m).wait()

    def scale(buf, nbp, nbase):
        @plsc.parallel_loop(0, SUB, 2, unroll=4)
        def r_body(r):
            for rr in range(2):
                s = nbp[pl.ds(nbase + r + rr, L)][0]
                for q in range(D // L):
                    buf[r + rr, pl.ds(q * L, L)] = (
                        buf[r + rr, pl.ds(q * L, L)] * s)

    def subs_run(row_base, nsub, p):
        sbp = sbA if p == 0 else sbB
        nbp = nbA if p == 0 else nbB
        d2p = d2A if p == 0 else d2B
        gbase = cidx * EROWS + row_base
        roff = gbase - (gbase // 8) * 8

        def a_body(g, _):
            sbp[pl.ds(g * L, L)] = sbp[pl.ds(g * L, L)] + voff
            return 0

        lax.fori_loop(0, nsub * SUB // L, a_body, 0)

        gd = [None] * nsub
        sd = [None] * nsub
        gd[0] = pltpu.async_copy(v.at[sbp.at[pl.ds(0, SUB)]], gA, sg0)
        for j in range(nsub):
            cur = gA if j % 2 == 0 else gB
            gd[j].wait()
            if j + 1 < nsub:
                if j - 1 >= 0:
                    sd[j - 1].wait()
                nxt = gB if j % 2 == 0 else gA
                sgn = sg1 if j % 2 == 0 else sg0
                gd[j + 1] = pltpu.async_copy(
                    v.at[sbp.at[pl.ds((j + 1) * SUB, SUB)]], nxt, sgn)
            scale(cur, nbp, j * SUB)
            ssem = ss0 if j % 2 == 0 else ss1
            sd[j] = pltpu.async_copy(cur, acc_sp.at[d2p.at[roff + j]], ssem,
                                     add=True)
        if nsub >= 2:
            sd[nsub - 2].wait()
        sd[nsub - 1].wait()

    def row_of(oc):
        return sid * ROWS_T + oc * O_SUB

    meta_issue(row_of(0), O_SUB, 0)

    def o_body(i, _):
        rb0 = row_of(2 * i)
        meta_wait(rb0, O_SUB, 0)
        meta_issue(row_of(2 * i + 1), O_SUB, 1)
        subs_run(rb0, O_SUB, 0)
        rb1 = row_of(2 * i + 1)
        meta_wait(rb1, O_SUB, 1)
        meta_issue(row_of(2 * i + 2), O_SUB, 0)
        subs_run(rb1, O_SUB, 1)
        return 0

    lax.fori_loop(0, (NOUT - 1) // 2, o_body, 0)
    meta_wait(row_of(NOUT - 1), O_SUB, 0)
    subs_run(row_of(NOUT - 1), O_SUB, 0)

    @pl.when(sid == NS - 1)
    def _():
        for m in meta_issue(XROW0, X_SUB, 1):
            m.wait()
        subs_run(XROW0, X_SUB, 1)

    plsc.subcore_barrier()

    if first:
        def f_body(j, _):
            ro = off_n + j * 2 * ECH
            pltpu.sync_copy(acc_sp.at[pl.ds(ro, 2 * ECH)], gA)
            pltpu.sync_copy(gA, out.at[pl.ds(voff + ro, 2 * ECH)])
            return 0

        lax.fori_loop(0, ROWS // (2 * ECH), f_body, 0)
    else:
        def e_body(j, _):
            ro = off_n + j * ECH
            pltpu.sync_copy(acc_sp.at[pl.ds(ro, ECH)], gA.at[pl.ds(0, ECH)])
            pltpu.sync_copy(t0.at[pl.ds(voff + ro, ECH)],
                            gA.at[pl.ds(ECH, ECH)])

            @plsc.parallel_loop(0, ECH * (D // L), 2, unroll=4)
            def c_body(i):
                for u in range(2):
                    r = (i + u) // (D // L)
                    c = ((i + u) % (D // L)) * L
                    a = gA[r, pl.ds(c, L)]
                    t = gA[ECH + r, pl.ds(c, L)]
                    gB[r, pl.ds(c, L)] = a + a - t

            pltpu.sync_copy(gB.at[pl.ds(0, ECH)],
                            out.at[pl.ds(voff + ro, ECH)])
            return 0

        lax.fori_loop(0, ROWS // ECH, e_body, 0)


def _prop_call(ei, nrm, eid2, v, t0, first):
    return pl.kernel(
        functools.partial(_prop_body, first=first),
        out_type=jax.ShapeDtypeStruct((NC * N, D), jnp.float32),
        mesh=_MESH,
        scratch_types=[
            pltpu.VMEM((OCHE,), jnp.int32),
            pltpu.VMEM((OCHE,), jnp.int32),
            pltpu.VMEM((OCHE + L,), jnp.float32),
            pltpu.VMEM((OCHE + L,), jnp.float32),
            pltpu.VMEM((O_SUB + 12, SUB), jnp.int32),
            pltpu.VMEM((O_SUB + 12, SUB), jnp.int32),
            pltpu.VMEM((SUB, D), jnp.float32),
            pltpu.VMEM((SUB, D), jnp.float32),
            pltpu.SemaphoreType.DMA,
            pltpu.SemaphoreType.DMA,
            pltpu.SemaphoreType.DMA,
            pltpu.SemaphoreType.DMA,
            pltpu.SemaphoreType.DMA,
            pltpu.VMEM_SHARED((N, D), jnp.float32),
        ],
    )(ei, nrm, eid2, v, t0)


_BLK = 1000


def _acc_body(t0_ref, t1_ref, w_ref, g_ref, o_ref):
    o_ref[...] = (g_ref[...]
                  + jnp.dot(t0_ref[...], w_ref[0],
                            preferred_element_type=jnp.float32)
                  + jnp.dot(t1_ref[...], w_ref[1],
                            preferred_element_type=jnp.float32))


def _acc(t, w2, g):
    nb = N // _BLK
    return pl.pallas_call(
        _acc_body,
        grid=(nb,),
        in_specs=[
            pl.BlockSpec((_BLK, D), lambda i: (i, 0)),
            pl.BlockSpec((_BLK, D), lambda i, nb=nb: (i + nb, 0)),
            pl.BlockSpec((2, D, 2 * HID), lambda i: (0, 0, 0)),
            pl.BlockSpec((_BLK, 2 * HID), lambda i: (i, 0)),
        ],
        out_specs=pl.BlockSpec((_BLK, 2 * HID), lambda i: (i, 0)),
        out_shape=jax.ShapeDtypeStruct((N, 2 * HID), jnp.float32),
    )(t, t, w2, g)


def _finish_body(g_ref, bzh_ref, wl_ref, bl_ref, o_ref):
    g = g_ref[...] + bzh_ref[...]
    z = jax.nn.sigmoid(g[:, :HID])
    ht = jnp.tanh(g[:, HID:])
    h = jax.nn.relu((1.0 - z) * ht)
    o_ref[...] = jnp.dot(h, wl_ref[...],
                         preferred_element_type=jnp.float32) + bl_ref[...]


def _finish(g, bzh, wl_pad, bl_pad):
    return pl.pallas_call(
        _finish_body,
        grid=(N // _BLK,),
        in_specs=[
            pl.BlockSpec((_BLK, 2 * HID), lambda i: (i, 0)),
            pl.BlockSpec((1, 2 * HID), lambda i: (0, 0)),
            pl.BlockSpec((HID, 128), lambda i: (0, 0)),
            pl.BlockSpec((1, 128), lambda i: (0, 0)),
        ],
        out_specs=pl.BlockSpec((_BLK, 128), lambda i: (i, 0)),
        out_shape=jax.ShapeDtypeStruct((N, 128), jnp.float32),
    )(g, bzh, wl_pad, bl_pad)


def kernel(x, edge_index, edge_weight, W_z, b_z, W_r, b_r, W_h, b_h, W_lin, b_lin):
    ei = edge_index.astype(jnp.int32).reshape(2 * E)
    w = edge_weight.astype(jnp.float32)

    nrm = _norm_call(ei, w)

    wz = W_z[:, :, :D, :]
    wh = W_h[:, :, :D, :]
    w_cat = jnp.concatenate([wz, wh], axis=-1)
    w_all = w_cat.transpose(1, 0, 2, 3)

    eid2 = jnp.concatenate(
        [ei.reshape(2 * EROWS, SUB), jnp.zeros((24, SUB), jnp.int32)])

    xx = jnp.concatenate([x, x], axis=0)
    g = _acc(xx, w_all[0], jnp.zeros((N, 2 * HID), jnp.float32))
    t1 = _prop_call(ei, nrm, eid2, xx, xx, True)
    g = _acc(t1, w_all[1], g)
    prev, cur = xx, t1
    for k in range(2, K):
        nxt = _prop_call(ei, nrm, eid2, cur, prev, False)
        g = _acc(nxt, w_all[k], g)
        prev, cur = cur, nxt

    bzh = jnp.concatenate([b_z, b_h]).reshape(1, 2 * HID)
    wl_pad = jnp.zeros((HID, 128), jnp.float32).at[:, :PRE_LEN].set(W_lin)
    bl_pad = jnp.zeros((1, 128), jnp.float32).at[0, :PRE_LEN].set(b_lin)

    out = _finish(g, bzh, wl_pad, bl_pad)
    return out[:, :PRE_LEN]


# speedup vs baseline: 14.4046x; 1.0402x over previous
import functools

import jax
import jax.numpy as jnp
from jax import lax
from jax.experimental.compute_on import compute_on
from jax.experimental import pallas as pl
from jax.experimental.pallas import tpu as pltpu
from jax.experimental.pallas import tpu_sc as plsc

N = 10000
E = 320000
D = 128
HID = 64
K = 16
PRE_LEN = 4

NC = 2
NS = 16
L = 16

EPT = E // NS
OCH = 2000
NOCH = EPT // OCH
SUB = 128
NSUB = 15
TAILSZ = OCH - NSUB * SUB
ROWS = 640
RSTEP = 624
ECH = 64
EROWS = E // SUB
O_SUB = 12
OCHE = O_SUB * SUB
ROWS_T = 156
NOUT = ROWS_T // O_SUB
XROW0 = NS * ROWS_T
X_SUB = EROWS - XROW0

_MESH = plsc.VectorSubcoreMesh(core_axis_name="c", subcore_axis_name="s",
                               num_cores=NC, num_subcores=NS)


def _zero_rows(buf, nrows):
    z = jnp.zeros((L,), jnp.float32)

    def body(i, _):
        r = i // (D // L)
        c = (i % (D // L)) * L
        buf[r, pl.ds(c, L)] = z
        return 0

    lax.fori_loop(0, nrows * (D // L), body, 0)


def _norm_body(ei, w, eid2, norm_out, deg_out, zb, wb, nb, dgb, d2, sem_m,
               sg0, sg1, ss0, ss1, deg_sp):
    cidx = lax.axis_index("c")
    sid = lax.axis_index("s")
    off_n = sid * RSTEP

    def zb_body(g, _):
        zb[pl.ds(g * L, L)] = jnp.zeros((L,), jnp.float32)
        return 0

    lax.fori_loop(0, ROWS // L, zb_body, 0)
    pltpu.sync_copy(zb.at[pl.ds(0, ROWS)], deg_sp.at[pl.ds(off_n, ROWS)])
    plsc.subcore_barrier()

    def meta(row_base, nsub):
        ebase = row_base * SUB
        gbase = cidx * EROWS + row_base
        base8 = (gbase // 8) * 8
        nrows = ((nsub + 15) // 8) * 8
        m1 = pltpu.async_copy(w.at[pl.ds(ebase, nsub * SUB)],
                              wb.at[pl.ds(0, nsub * SUB)], sem_m)
        m2 = pltpu.async_copy(eid2.at[pl.ds(base8, nrows)],
                              d2.at[pl.ds(0, nrows)], sem_m)
        m1.wait()
        m2.wait()
        return gbase - base8

    def scat_outer(row_base, nsub):
        roff = meta(row_base, nsub)
        sd = [None] * nsub
        for j in range(nsub):
            ssem = ss0 if j % 2 == 0 else ss1
            sd[j] = pltpu.async_copy(wb.at[pl.ds(j * SUB, SUB)],
                                     deg_sp.at[d2.at[roff + j]], ssem,
                                     add=True)
            if j - 2 >= 0:
                sd[j - 2].wait()
        for j in range(max(0, nsub - 2), nsub):
            sd[j].wait()

    def row_of(oc):
        return sid * ROWS_T + oc * O_SUB

    def p1_body(oc, _):
        scat_outer(row_of(oc), O_SUB)
        return 0

    lax.fori_loop(0, NOUT, p1_body, 0)

    @pl.when(sid == NS - 1)
    def _():
        scat_outer(XROW0, X_SUB)

    plsc.subcore_barrier()

    pltpu.sync_copy(deg_sp.at[pl.ds(off_n, ROWS)], zb.at[pl.ds(0, ROWS)])
    pltpu.sync_copy(zb.at[pl.ds(0, ROWS)],
                    deg_out.at[pl.ds(cidx * N + off_n, ROWS)])
    plsc.subcore_barrier()

    def norm_outer(row_base, nsub):
        roff = meta(row_base, nsub)
        nrows = ((nsub + 15) // 8) * 8

        def adj_body(g, _):
            r = g // (SUB // L)
            c = (g % (SUB // L)) * L
            d2[r, pl.ds(c, L)] = d2[r, pl.ds(c, L)] + cidx * N
            return 0

        lax.fori_loop(0, nrows * SUB // L, adj_body, 0)

        gd = [None] * nsub
        for j in range(nsub):
            gsem = sg0 if j % 2 == 0 else sg1
            gd[j] = pltpu.async_copy(deg_out.at[d2.at[roff + j]],
                                     dgb.at[pl.ds(j * SUB, SUB)], gsem)
            if j - 2 >= 0:
                gd[j - 2].wait()
        for j in range(max(0, nsub - 2), nsub):
            gd[j].wait()

        @plsc.parallel_loop(0, nsub * SUB // L, 1, unroll=4)
        def div_body(g):
            nb[pl.ds(g * L, L)] = (wb[pl.ds(g * L, L)]
                                   / dgb[pl.ds(g * L, L)])

        pltpu.sync_copy(nb.at[pl.ds(0, nsub * SUB)],
                        norm_out.at[pl.ds(cidx * E + row_base * SUB,
                                          nsub * SUB)])

    def p3_body(oc, _):
        norm_outer(row_of(oc), O_SUB)
        return 0

    lax.fori_loop(0, NOUT, p3_body, 0)

    @pl.when(sid == NS - 1)
    def _():
        norm_outer(XROW0, X_SUB)


def _norm_call(ei, w, eid2):
    norm, _deg = pl.kernel(
        _norm_body,
        out_type=(jax.ShapeDtypeStruct((NC * E,), jnp.float32),
                  jax.ShapeDtypeStruct((NC * N,), jnp.float32)),
        mesh=_MESH,
        scratch_types=[
            pltpu.VMEM((ROWS,), jnp.float32),
            pltpu.VMEM((OCHE,), jnp.float32),
            pltpu.VMEM((OCHE,), jnp.float32),
            pltpu.VMEM((OCHE,), jnp.float32),
            pltpu.VMEM((O_SUB + 12, SUB), jnp.int32),
            pltpu.SemaphoreType.DMA,
            pltpu.SemaphoreType.DMA,
            pltpu.SemaphoreType.DMA,
            pltpu.SemaphoreType.DMA,
            pltpu.SemaphoreType.DMA,
            pltpu.VMEM_SHARED((N,), jnp.float32),
        ],
    )(ei, w, eid2)
    return norm


def _prop_body(ei, nrm, eid2, v, t0, out, sbA, sbB, nbA, nbB, d2A, d2B, gA,
               gB, sem_m, sg0, sg1, ss0, ss1, acc_sp, *, first):
    cidx = lax.axis_index("c")
    sid = lax.axis_index("s")
    off_n = sid * RSTEP
    voff = cidx * N

    _zero_rows(gA, ECH)
    zd = [pltpu.async_copy(gA.at[pl.ds(0, ECH)],
                           acc_sp.at[pl.ds(off_n + j * ECH, ECH)], sem_m)
          for j in range(ROWS // ECH)]
    for d in zd:
        d.wait()
    plsc.subcore_barrier()

    def scale(buf, nbase):
        @plsc.parallel_loop(0, SUB, 2, unroll=4)
        def r_body(r):
            for rr in range(2):
                s = nb[pl.ds(nbase + r + rr, L)][0]
                for q in range(D // L):
                    buf[r + rr, pl.ds(q * L, L)] = (
                        buf[r + rr, pl.ds(q * L, L)] * s)

    def meta_issue(row_base, nsub, p):
        ebase = row_base * SUB
        gbase = cidx * EROWS + row_base
        base8 = (gbase // 8) * 8
        nrows = ((nsub + 15) // 8) * 8
        nbp = nbA if p == 0 else nbB
        sbp = sbA if p == 0 else sbB
        d2p = d2A if p == 0 else d2B
        m1 = pltpu.async_copy(nrm.at[pl.ds(cidx * E + ebase, nsub * SUB)],
                              nbp.at[pl.ds(0, nsub * SUB)], sem_m)
        m2 = pltpu.async_copy(ei.at[pl.ds((1 - cidx) * E + ebase,
                                          nsub * SUB)],
                              sbp.at[pl.ds(0, nsub * SUB)], sem_m)
        m3 = pltpu.async_copy(eid2.at[pl.ds(base8, nrows)],
                              d2p.at[pl.ds(0, nrows)], sem_m)
        return (m1, m2, m3)

    def meta_wait(row_base, nsub, p):
        ebase = row_base * SUB
        gbase = cidx * EROWS + row_base
        base8 = (gbase // 8) * 8
        nrows = ((nsub + 15) // 8) * 8
        nbp = nbA if p == 0 else nbB
        sbp = sbA if p == 0 else sbB
        d2p = d2A if p == 0 else d2B
        pltpu.make_async_copy(nrm.at[pl.ds(cidx * E + ebase, nsub * SUB)],
                              nbp.at[pl.ds(0, nsub * SUB)], sem_m).wait()
        pltpu.make_async_copy(ei.at[pl.ds((1 - cidx) * E + ebase,
                                          nsub * SUB)],
                              sbp.at[pl.ds(0, nsub * SUB)], sem_m).wait()
        pltpu.make_async_copy(eid2.at[pl.ds(base8, nrows)],
                              d2p.at[pl.ds(0, nrows)], sem_m).wait()

    def scale(buf, nbp, nbase):
        @plsc.parallel_loop(0, SUB, 2, unroll=4)
        def r_body(r):
            for rr in range(2):
                s = nbp[pl.ds(nbase + r + rr, L)][0]
                for q in range(D // L):
                    buf[r + rr, pl.ds(q * L, L)] = (
                        buf[r + rr, pl.ds(q * L, L)] * s)

    def subs_run(row_base, nsub, p):
        sbp = sbA if p == 0 else sbB
        nbp = nbA if p == 0 else nbB
        d2p = d2A if p == 0 else d2B
        gbase = cidx * EROWS + row_base
        roff = gbase - (gbase // 8) * 8

        def a_body(g, _):
            sbp[pl.ds(g * L, L)] = sbp[pl.ds(g * L, L)] + voff
            return 0

        lax.fori_loop(0, nsub * SUB // L, a_body, 0)

        gd = [None] * nsub
        sd = [None] * nsub
        gd[0] = pltpu.async_copy(v.at[sbp.at[pl.ds(0, SUB)]], gA, sg0)
        for j in range(nsub):
            cur = gA if j % 2 == 0 else gB
            gd[j].wait()
            if j + 1 < nsub:
                if j - 1 >= 0:
                    sd[j - 1].wait()
                nxt = gB if j % 2 == 0 else gA
                sgn = sg1 if j % 2 == 0 else sg0
                gd[j + 1] = pltpu.async_copy(
                    v.at[sbp.at[pl.ds((j + 1) * SUB, SUB)]], nxt, sgn)
            scale(cur, nbp, j * SUB)
            ssem = ss0 if j % 2 == 0 else ss1
            sd[j] = pltpu.async_copy(cur, acc_sp.at[d2p.at[roff + j]], ssem,
                                     add=True)
        if nsub >= 2:
            sd[nsub - 2].wait()
        sd[nsub - 1].wait()

    def row_of(oc):
        return sid * ROWS_T + oc * O_SUB

    meta_issue(row_of(0), O_SUB, 0)

    def o_body(i, _):
        rb0 = row_of(2 * i)
        meta_wait(rb0, O_SUB, 0)
        meta_issue(row_of(2 * i + 1), O_SUB, 1)
        subs_run(rb0, O_SUB, 0)
        rb1 = row_of(2 * i + 1)
        meta_wait(rb1, O_SUB, 1)
        meta_issue(row_of(2 * i + 2), O_SUB, 0)
        subs_run(rb1, O_SUB, 1)
        return 0

    lax.fori_loop(0, (NOUT - 1) // 2, o_body, 0)
    meta_wait(row_of(NOUT - 1), O_SUB, 0)
    subs_run(row_of(NOUT - 1), O_SUB, 0)

    @pl.when(sid == NS - 1)
    def _():
        for m in meta_issue(XROW0, X_SUB, 1):
            m.wait()
        subs_run(XROW0, X_SUB, 1)

    plsc.subcore_barrier()

    if first:
        def f_body(j, _):
            ro = off_n + j * 2 * ECH
            pltpu.sync_copy(acc_sp.at[pl.ds(ro, 2 * ECH)], gA)
            pltpu.sync_copy(gA, out.at[pl.ds(voff + ro, 2 * ECH)])
            return 0

        lax.fori_loop(0, ROWS // (2 * ECH), f_body, 0)
    else:
        def e_body(j, _):
            ro = off_n + j * ECH
            pltpu.sync_copy(acc_sp.at[pl.ds(ro, ECH)], gA.at[pl.ds(0, ECH)])
            pltpu.sync_copy(t0.at[pl.ds(voff + ro, ECH)],
                            gA.at[pl.ds(ECH, ECH)])

            @plsc.parallel_loop(0, ECH * (D // L), 2, unroll=4)
            def c_body(i):
                for u in range(2):
                    r = (i + u) // (D // L)
                    c = ((i + u) % (D // L)) * L
                    a = gA[r, pl.ds(c, L)]
                    t = gA[ECH + r, pl.ds(c, L)]
                    gB[r, pl.ds(c, L)] = a + a - t

            pltpu.sync_copy(gB.at[pl.ds(0, ECH)],
                            out.at[pl.ds(voff + ro, ECH)])
            return 0

        lax.fori_loop(0, ROWS // ECH, e_body, 0)


def _prop_call(ei, nrm, eid2, v, t0, first):
    return pl.kernel(
        functools.partial(_prop_body, first=first),
        out_type=jax.ShapeDtypeStruct((NC * N, D), jnp.float32),
        mesh=_MESH,
        scratch_types=[
            pltpu.VMEM((OCHE,), jnp.int32),
            pltpu.VMEM((OCHE,), jnp.int32),
            pltpu.VMEM((OCHE + L,), jnp.float32),
            pltpu.VMEM((OCHE + L,), jnp.float32),
            pltpu.VMEM((O_SUB + 12, SUB), jnp.int32),
            pltpu.VMEM((O_SUB + 12, SUB), jnp.int32),
            pltpu.VMEM((SUB, D), jnp.float32),
            pltpu.VMEM((SUB, D), jnp.float32),
            pltpu.SemaphoreType.DMA,
            pltpu.SemaphoreType.DMA,
            pltpu.SemaphoreType.DMA,
            pltpu.SemaphoreType.DMA,
            pltpu.SemaphoreType.DMA,
            pltpu.VMEM_SHARED((N, D), jnp.float32),
        ],
    )(ei, nrm, eid2, v, t0)


_BLK = 1000


def _acc_body(t0_ref, t1_ref, w_ref, g_ref, o_ref):
    o_ref[...] = (g_ref[...]
                  + jnp.dot(t0_ref[...], w_ref[0],
                            preferred_element_type=jnp.float32)
                  + jnp.dot(t1_ref[...], w_ref[1],
                            preferred_element_type=jnp.float32))


def _acc(t, w2, g):
    nb = N // _BLK
    return pl.pallas_call(
        _acc_body,
        grid=(nb,),
        in_specs=[
            pl.BlockSpec((_BLK, D), lambda i: (i, 0)),
            pl.BlockSpec((_BLK, D), lambda i, nb=nb: (i + nb, 0)),
            pl.BlockSpec((2, D, 2 * HID), lambda i: (0, 0, 0)),
            pl.BlockSpec((_BLK, 2 * HID), lambda i: (i, 0)),
        ],
        out_specs=pl.BlockSpec((_BLK, 2 * HID), lambda i: (i, 0)),
        out_shape=jax.ShapeDtypeStruct((N, 2 * HID), jnp.float32),
    )(t, t, w2, g)


def _finish_body(g_ref, bzh_ref, wl_ref, bl_ref, o_ref):
    g = g_ref[...] + bzh_ref[...]
    z = jax.nn.sigmoid(g[:, :HID])
    ht = jnp.tanh(g[:, HID:])
    h = jax.nn.relu((1.0 - z) * ht)
    o_ref[...] = jnp.dot(h, wl_ref[...],
                         preferred_element_type=jnp.float32) + bl_ref[...]


def _finish(g, bzh, wl_pad, bl_pad):
    return pl.pallas_call(
        _finish_body,
        grid=(N // _BLK,),
        in_specs=[
            pl.BlockSpec((_BLK, 2 * HID), lambda i: (i, 0)),
            pl.BlockSpec((1, 2 * HID), lambda i: (0, 0)),
            pl.BlockSpec((HID, 128), lambda i: (0, 0)),
            pl.BlockSpec((1, 128), lambda i: (0, 0)),
        ],
        out_specs=pl.BlockSpec((_BLK, 128), lambda i: (i, 0)),
        out_shape=jax.ShapeDtypeStruct((N, 128), jnp.float32),
    )(g, bzh, wl_pad, bl_pad)


def kernel(x, edge_index, edge_weight, W_z, b_z, W_r, b_r, W_h, b_h, W_lin, b_lin):
    ei = edge_index.astype(jnp.int32).reshape(2 * E)
    w = edge_weight.astype(jnp.float32)


    wz = W_z[:, :, :D, :]
    wh = W_h[:, :, :D, :]
    w_cat = jnp.concatenate([wz, wh], axis=-1)
    w_all = w_cat.transpose(1, 0, 2, 3)

    eid2 = jnp.concatenate(
        [ei.reshape(2 * EROWS, SUB), jnp.zeros((24, SUB), jnp.int32)])
    nrm = _norm_call(ei, w, eid2)

    xx = jnp.concatenate([x, x], axis=0)
    g = _acc(xx, w_all[0], jnp.zeros((N, 2 * HID), jnp.float32))
    t1 = _prop_call(ei, nrm, eid2, xx, xx, True)
    g = _acc(t1, w_all[1], g)
    prev, cur = xx, t1
    for k in range(2, K):
        nxt = _prop_call(ei, nrm, eid2, cur, prev, False)
        g = _acc(nxt, w_all[k], g)
        prev, cur = cur, nxt

    bzh = jnp.concatenate([b_z, b_h]).reshape(1, 2 * HID)
    wl_pad = jnp.zeros((HID, 128), jnp.float32).at[:, :PRE_LEN].set(W_lin)
    bl_pad = jnp.zeros((1, 128), jnp.float32).at[0, :PRE_LEN].set(b_lin)

    out = _finish(g, bzh, wl_pad, bl_pad)
    return out[:, :PRE_LEN]


# speedup vs baseline: 14.6600x; 1.0177x over previous
import functools

import jax
import jax.numpy as jnp
from jax import lax
from jax.experimental.compute_on import compute_on
from jax.experimental import pallas as pl
from jax.experimental.pallas import tpu as pltpu
from jax.experimental.pallas import tpu_sc as plsc

N = 10000
E = 320000
D = 128
HID = 64
K = 16
PRE_LEN = 4

NC = 2
NS = 16
L = 16

EPT = E // NS
OCH = 2000
NOCH = EPT // OCH
SUB = 128
NSUB = 15
TAILSZ = OCH - NSUB * SUB
ROWS = 640
RSTEP = 624
ECH = 64
EROWS = E // SUB
O_SUB = 12
OCHE = O_SUB * SUB
ROWS_T = 156
NOUT = ROWS_T // O_SUB
XROW0 = NS * ROWS_T
X_SUB = EROWS - XROW0

_MESH = plsc.VectorSubcoreMesh(core_axis_name="c", subcore_axis_name="s",
                               num_cores=NC, num_subcores=NS)


def _zero_rows(buf, nrows):
    z = jnp.zeros((L,), jnp.float32)

    def body(i, _):
        r = i // (D // L)
        c = (i % (D // L)) * L
        buf[r, pl.ds(c, L)] = z
        return 0

    lax.fori_loop(0, nrows * (D // L), body, 0)


def _norm_body(ei, w, eid2, norm_out, deg_out, zb, wb, nb, dgb, d2, sem_m,
               sg0, sg1, ss0, ss1, deg_sp):
    cidx = lax.axis_index("c")
    sid = lax.axis_index("s")
    off_n = sid * RSTEP

    def zb_body(g, _):
        zb[pl.ds(g * L, L)] = jnp.zeros((L,), jnp.float32)
        return 0

    lax.fori_loop(0, ROWS // L, zb_body, 0)
    pltpu.sync_copy(zb.at[pl.ds(0, ROWS)], deg_sp.at[pl.ds(off_n, ROWS)])
    plsc.subcore_barrier()

    def meta(row_base, nsub):
        ebase = row_base * SUB
        gbase = cidx * EROWS + row_base
        base8 = (gbase // 8) * 8
        nrows = ((nsub + 15) // 8) * 8
        m1 = pltpu.async_copy(w.at[pl.ds(ebase, nsub * SUB)],
                              wb.at[pl.ds(0, nsub * SUB)], sem_m)
        m2 = pltpu.async_copy(eid2.at[pl.ds(base8, nrows)],
                              d2.at[pl.ds(0, nrows)], sem_m)
        m1.wait()
        m2.wait()
        return gbase - base8

    def scat_outer(row_base, nsub):
        roff = meta(row_base, nsub)
        sd = [None] * nsub
        for j in range(nsub):
            ssem = ss0 if j % 2 == 0 else ss1
            sd[j] = pltpu.async_copy(wb.at[pl.ds(j * SUB, SUB)],
                                     deg_sp.at[d2.at[roff + j]], ssem,
                                     add=True)
            if j - 2 >= 0:
                sd[j - 2].wait()
        for j in range(max(0, nsub - 2), nsub):
            sd[j].wait()

    def row_of(oc):
        return sid * ROWS_T + oc * O_SUB

    def p1_body(oc, _):
        scat_outer(row_of(oc), O_SUB)
        return 0

    lax.fori_loop(0, NOUT, p1_body, 0)

    @pl.when(sid == NS - 1)
    def _():
        scat_outer(XROW0, X_SUB)

    plsc.subcore_barrier()

    pltpu.sync_copy(deg_sp.at[pl.ds(off_n, ROWS)], zb.at[pl.ds(0, ROWS)])
    pltpu.sync_copy(zb.at[pl.ds(0, ROWS)],
                    deg_out.at[pl.ds(cidx * N + off_n, ROWS)])
    plsc.subcore_barrier()

    def norm_outer(row_base, nsub):
        roff = meta(row_base, nsub)
        nrows = ((nsub + 15) // 8) * 8

        def adj_body(g, _):
            r = g // (SUB // L)
            c = (g % (SUB // L)) * L
            d2[r, pl.ds(c, L)] = d2[r, pl.ds(c, L)] + cidx * N
            return 0

        lax.fori_loop(0, nrows * SUB // L, adj_body, 0)

        gd = [None] * nsub
        for j in range(nsub):
            gsem = sg0 if j % 2 == 0 else sg1
            gd[j] = pltpu.async_copy(deg_out.at[d2.at[roff + j]],
                                     dgb.at[pl.ds(j * SUB, SUB)], gsem)
            if j - 2 >= 0:
                gd[j - 2].wait()
        for j in range(max(0, nsub - 2), nsub):
            gd[j].wait()

        @plsc.parallel_loop(0, nsub * SUB // L, 1, unroll=4)
        def div_body(g):
            nb[pl.ds(g * L, L)] = (wb[pl.ds(g * L, L)]
                                   / dgb[pl.ds(g * L, L)])

        pltpu.sync_copy(nb.at[pl.ds(0, nsub * SUB)],
                        norm_out.at[pl.ds(cidx * E + row_base * SUB,
                                          nsub * SUB)])

    def p3_body(oc, _):
        norm_outer(row_of(oc), O_SUB)
        return 0

    lax.fori_loop(0, NOUT, p3_body, 0)

    @pl.when(sid == NS - 1)
    def _():
        norm_outer(XROW0, X_SUB)


def _norm_call(ei, w, eid2):
    norm, _deg = pl.kernel(
        _norm_body,
        out_type=(jax.ShapeDtypeStruct((NC * E,), jnp.float32),
                  jax.ShapeDtypeStruct((NC * N,), jnp.float32)),
        mesh=_MESH,
        scratch_types=[
            pltpu.VMEM((ROWS,), jnp.float32),
            pltpu.VMEM((OCHE,), jnp.float32),
            pltpu.VMEM((OCHE,), jnp.float32),
            pltpu.VMEM((OCHE,), jnp.float32),
            pltpu.VMEM((O_SUB + 12, SUB), jnp.int32),
            pltpu.SemaphoreType.DMA,
            pltpu.SemaphoreType.DMA,
            pltpu.SemaphoreType.DMA,
            pltpu.SemaphoreType.DMA,
            pltpu.SemaphoreType.DMA,
            pltpu.VMEM_SHARED((N,), jnp.float32),
        ],
    )(ei, w, eid2)
    return norm


def _prop_body(ei, nrm, eid2, v, t0, out, sbA, sbB, nbA, nbB, d2A, d2B, gA,
               gB, sem_m, sg0, sg1, ss0, ss1, acc_sp, *, first):
    cidx = lax.axis_index("c")
    sid = lax.axis_index("s")
    off_n = sid * RSTEP
    voff = cidx * N

    _zero_rows(gA, ECH)
    zd = [pltpu.async_copy(gA.at[pl.ds(0, ECH)],
                           acc_sp.at[pl.ds(off_n + j * ECH, ECH)], sem_m)
          for j in range(ROWS // ECH)]
    for d in zd:
        d.wait()
    plsc.subcore_barrier()

    def scale(buf, nbase):
        @plsc.parallel_loop(0, SUB, 2, unroll=4)
        def r_body(r):
            for rr in range(2):
                s = nb[pl.ds(nbase + r + rr, L)][0]
                for q in range(D // L):
                    buf[r + rr, pl.ds(q * L, L)] = (
                        buf[r + rr, pl.ds(q * L, L)] * s)

    def meta_issue(row_base, nsub, p):
        ebase = row_base * SUB
        gbase = cidx * EROWS + row_base
        base8 = (gbase // 8) * 8
        nrows = ((nsub + 15) // 8) * 8
        nbp = nbA if p == 0 else nbB
        sbp = sbA if p == 0 else sbB
        d2p = d2A if p == 0 else d2B
        m1 = pltpu.async_copy(nrm.at[pl.ds(cidx * E + ebase, nsub * SUB)],
                              nbp.at[pl.ds(0, nsub * SUB)], sem_m)
        m2 = pltpu.async_copy(ei.at[pl.ds((1 - cidx) * E + ebase,
                                          nsub * SUB)],
                              sbp.at[pl.ds(0, nsub * SUB)], sem_m)
        m3 = pltpu.async_copy(eid2.at[pl.ds(base8, nrows)],
                              d2p.at[pl.ds(0, nrows)], sem_m)
        return (m1, m2, m3)

    def meta_wait(row_base, nsub, p):
        ebase = row_base * SUB
        gbase = cidx * EROWS + row_base
        base8 = (gbase // 8) * 8
        nrows = ((nsub + 15) // 8) * 8
        nbp = nbA if p == 0 else nbB
        sbp = sbA if p == 0 else sbB
        d2p = d2A if p == 0 else d2B
        pltpu.make_async_copy(nrm.at[pl.ds(cidx * E + ebase, nsub * SUB)],
                              nbp.at[pl.ds(0, nsub * SUB)], sem_m).wait()
        pltpu.make_async_copy(ei.at[pl.ds((1 - cidx) * E + ebase,
                                          nsub * SUB)],
                              sbp.at[pl.ds(0, nsub * SUB)], sem_m).wait()
        pltpu.make_async_copy(eid2.at[pl.ds(base8, nrows)],
                              d2p.at[pl.ds(0, nrows)], sem_m).wait()

    def scale(buf, nbp, nbase):
        @plsc.parallel_loop(0, SUB, 2, unroll=4)
        def r_body(r):
            for rr in range(2):
                s = nbp[pl.ds(nbase + r + rr, L)][0]
                for q in range(D // L):
                    buf[r + rr, pl.ds(q * L, L)] = (
                        buf[r + rr, pl.ds(q * L, L)] * s)

    def subs_run(row_base, nsub, p):
        sbp = sbA if p == 0 else sbB
        nbp = nbA if p == 0 else nbB
        d2p = d2A if p == 0 else d2B
        gbase = cidx * EROWS + row_base
        roff = gbase - (gbase // 8) * 8

        def a_body(g, _):
            sbp[pl.ds(g * L, L)] = sbp[pl.ds(g * L, L)] + voff
            return 0

        lax.fori_loop(0, nsub * SUB // L, a_body, 0)

        gd = [None] * nsub
        sd = [None] * nsub
        gd[0] = pltpu.async_copy(v.at[sbp.at[pl.ds(0, SUB)]], gA, sg0)
        for j in range(nsub):
            cur = gA if j % 2 == 0 else gB
            gd[j].wait()
            if j + 1 < nsub:
                if j - 1 >= 0:
                    sd[j - 1].wait()
                nxt = gB if j % 2 == 0 else gA
                sgn = sg1 if j % 2 == 0 else sg0
                gd[j + 1] = pltpu.async_copy(
                    v.at[sbp.at[pl.ds((j + 1) * SUB, SUB)]], nxt, sgn)
            scale(cur, nbp, j * SUB)
            ssem = ss0 if j % 2 == 0 else ss1
            sd[j] = pltpu.async_copy(cur, acc_sp.at[d2p.at[roff + j]], ssem,
                                     add=True)
        if nsub >= 2:
            sd[nsub - 2].wait()
        sd[nsub - 1].wait()

    def row_of(oc):
        return sid * ROWS_T + oc * O_SUB

    meta_issue(row_of(0), O_SUB, 0)

    def o_body(i, _):
        rb0 = row_of(2 * i)
        meta_wait(rb0, O_SUB, 0)
        meta_issue(row_of(2 * i + 1), O_SUB, 1)
        subs_run(rb0, O_SUB, 0)
        rb1 = row_of(2 * i + 1)
        meta_wait(rb1, O_SUB, 1)
        meta_issue(row_of(2 * i + 2), O_SUB, 0)
        subs_run(rb1, O_SUB, 1)
        return 0

    lax.fori_loop(0, (NOUT - 1) // 2, o_body, 0)
    meta_wait(row_of(NOUT - 1), O_SUB, 0)
    subs_run(row_of(NOUT - 1), O_SUB, 0)

    @pl.when(sid == NS - 1)
    def _():
        for m in meta_issue(XROW0, X_SUB, 1):
            m.wait()
        subs_run(XROW0, X_SUB, 1)

    plsc.subcore_barrier()

    if first:
        def f_body(j, _):
            ro = off_n + j * 2 * ECH
            pltpu.sync_copy(acc_sp.at[pl.ds(ro, 2 * ECH)], gA)
            pltpu.sync_copy(gA, out.at[pl.ds(voff + ro, 2 * ECH)])
            return 0

        lax.fori_loop(0, ROWS // (2 * ECH), f_body, 0)
    else:
        def e_body(j, _):
            ro = off_n + j * SUB
            pltpu.sync_copy(acc_sp.at[pl.ds(ro, SUB)], gA)
            pltpu.sync_copy(t0.at[pl.ds(voff + ro, SUB)], gB)

            @plsc.parallel_loop(0, SUB * (D // L), 2, unroll=4)
            def c_body(i):
                for u in range(2):
                    r = (i + u) // (D // L)
                    c = ((i + u) % (D // L)) * L
                    a = gA[r, pl.ds(c, L)]
                    t = gB[r, pl.ds(c, L)]
                    gA[r, pl.ds(c, L)] = a + a - t

            pltpu.sync_copy(gA, out.at[pl.ds(voff + ro, SUB)])
            return 0

        lax.fori_loop(0, ROWS // SUB, e_body, 0)


def _prop_call(ei, nrm, eid2, v, t0, first):
    return pl.kernel(
        functools.partial(_prop_body, first=first),
        out_type=jax.ShapeDtypeStruct((NC * N, D), jnp.float32),
        mesh=_MESH,
        scratch_types=[
            pltpu.VMEM((OCHE,), jnp.int32),
            pltpu.VMEM((OCHE,), jnp.int32),
            pltpu.VMEM((OCHE + L,), jnp.float32),
            pltpu.VMEM((OCHE + L,), jnp.float32),
            pltpu.VMEM((O_SUB + 12, SUB), jnp.int32),
            pltpu.VMEM((O_SUB + 12, SUB), jnp.int32),
            pltpu.VMEM((SUB, D), jnp.float32),
            pltpu.VMEM((SUB, D), jnp.float32),
            pltpu.SemaphoreType.DMA,
            pltpu.SemaphoreType.DMA,
            pltpu.SemaphoreType.DMA,
            pltpu.SemaphoreType.DMA,
            pltpu.SemaphoreType.DMA,
            pltpu.VMEM_SHARED((N, D), jnp.float32),
        ],
    )(ei, nrm, eid2, v, t0)


_BLK = 1000


def _acc_body(t0_ref, t1_ref, w_ref, g_ref, o_ref):
    o_ref[...] = (g_ref[...]
                  + jnp.dot(t0_ref[...], w_ref[0],
                            preferred_element_type=jnp.float32)
                  + jnp.dot(t1_ref[...], w_ref[1],
                            preferred_element_type=jnp.float32))


def _acc(t, w2, g):
    nb = N // _BLK
    return pl.pallas_call(
        _acc_body,
        grid=(nb,),
        in_specs=[
            pl.BlockSpec((_BLK, D), lambda i: (i, 0)),
            pl.BlockSpec((_BLK, D), lambda i, nb=nb: (i + nb, 0)),
            pl.BlockSpec((2, D, 2 * HID), lambda i: (0, 0, 0)),
            pl.BlockSpec((_BLK, 2 * HID), lambda i: (i, 0)),
        ],
        out_specs=pl.BlockSpec((_BLK, 2 * HID), lambda i: (i, 0)),
        out_shape=jax.ShapeDtypeStruct((N, 2 * HID), jnp.float32),
    )(t, t, w2, g)


def _finish_body(g_ref, bzh_ref, wl_ref, bl_ref, o_ref):
    g = g_ref[...] + bzh_ref[...]
    z = jax.nn.sigmoid(g[:, :HID])
    ht = jnp.tanh(g[:, HID:])
    h = jax.nn.relu((1.0 - z) * ht)
    o_ref[...] = jnp.dot(h, wl_ref[...],
                         preferred_element_type=jnp.float32) + bl_ref[...]


def _finish(g, bzh, wl_pad, bl_pad):
    return pl.pallas_call(
        _finish_body,
        grid=(N // _BLK,),
        in_specs=[
            pl.BlockSpec((_BLK, 2 * HID), lambda i: (i, 0)),
            pl.BlockSpec((1, 2 * HID), lambda i: (0, 0)),
            pl.BlockSpec((HID, 128), lambda i: (0, 0)),
            pl.BlockSpec((1, 128), lambda i: (0, 0)),
        ],
        out_specs=pl.BlockSpec((_BLK, 128), lambda i: (i, 0)),
        out_shape=jax.ShapeDtypeStruct((N, 128), jnp.float32),
    )(g, bzh, wl_pad, bl_pad)


def kernel(x, edge_index, edge_weight, W_z, b_z, W_r, b_r, W_h, b_h, W_lin, b_lin):
    ei = edge_index.astype(jnp.int32).reshape(2 * E)
    w = edge_weight.astype(jnp.float32)


    wz = W_z[:, :, :D, :]
    wh = W_h[:, :, :D, :]
    w_cat = jnp.concatenate([wz, wh], axis=-1)
    w_all = w_cat.transpose(1, 0, 2, 3)

    eid2 = jnp.concatenate(
        [ei.reshape(2 * EROWS, SUB), jnp.zeros((24, SUB), jnp.int32)])
    nrm = _norm_call(ei, w, eid2)

    xx = jnp.concatenate([x, x], axis=0)
    g = _acc(xx, w_all[0], jnp.zeros((N, 2 * HID), jnp.float32))
    t1 = _prop_call(ei, nrm, eid2, xx, xx, True)
    g = _acc(t1, w_all[1], g)
    prev, cur = xx, t1
    for k in range(2, K):
        nxt = _prop_call(ei, nrm, eid2, cur, prev, False)
        g = _acc(nxt, w_all[k], g)
        prev, cur = cur, nxt

    bzh = jnp.concatenate([b_z, b_h]).reshape(1, 2 * HID)
    wl_pad = jnp.zeros((HID, 128), jnp.float32).at[:, :PRE_LEN].set(W_lin)
    bl_pad = jnp.zeros((1, 128), jnp.float32).at[0, :PRE_LEN].set(b_lin)

    out = _finish(g, bzh, wl_pad, bl_pad)
    return out[:, :PRE_LEN]
